# baseline (speedup 1.0000x reference)
import functools
import math

import jax
import jax.numpy as jnp
from jax import lax
from jax.experimental import pallas as pl
from jax.experimental.pallas import tpu as pltpu
from jax.experimental.pallas import tpu_sc as plsc

F32 = jnp.float32
NW = 32
SUB = 128
BE = 1280


def _sc_mesh():
    return plsc.VectorSubcoreMesh(core_axis_name="c", subcore_axis_name="s")


_SC_PARAMS = pltpu.CompilerParams(needs_layout_passes=False)


def _make_sc_coorddiff(N_ACC, E_pad, EW):
    nsub_c = EW // 16

    @functools.partial(
        pl.kernel,
        out_type=[
            jax.ShapeDtypeStruct((E_pad,), F32),
            jax.ShapeDtypeStruct((E_pad,), F32),
            jax.ShapeDtypeStruct((E_pad,), F32),
        ],
        mesh=_sc_mesh(),
        scratch_types=[
            pltpu.VMEM((N_ACC * 4,), F32),
            pltpu.VMEM((EW,), jnp.int32),
            pltpu.VMEM((EW,), jnp.int32),
            pltpu.VMEM((EW,), F32),
            pltpu.VMEM((EW,), F32),
            pltpu.VMEM((EW,), F32),
        ],
        compiler_params=_SC_PARAMS,
    )
    def sc_coorddiff(cflat_hbm, src_hbm, dst_hbm,
                     dx_out, dy_out, dz_out,
                     cflat_v, srcv, dstv, dxb, dyb, dzb):
        cid = lax.axis_index("c")
        sid = lax.axis_index("s")
        wid = sid * 2 + cid
        base = wid * EW

        pltpu.sync_copy(cflat_hbm, cflat_v)
        pltpu.sync_copy(src_hbm.at[pl.ds(base, EW)], srcv)
        pltpu.sync_copy(dst_hbm.at[pl.ds(base, EW)], dstv)

        def cbody(k, carry):
            off = k * 16
            s16 = srcv[pl.ds(off, 16)] * 4
            d16 = dstv[pl.ds(off, 16)] * 4
            for c, buf in ((0, dxb), (1, dyb), (2, dzb)):
                a = plsc.load_gather(cflat_v, [s16 + c])
                b = plsc.load_gather(cflat_v, [d16 + c])
                buf[pl.ds(off, 16)] = a - b
            return carry

        lax.fori_loop(0, nsub_c, cbody, 0)
        pltpu.sync_copy(dxb, dx_out.at[pl.ds(base, EW)])
        pltpu.sync_copy(dyb, dy_out.at[pl.ds(base, EW)])
        pltpu.sync_copy(dzb, dz_out.at[pl.ds(base, EW)])

    return sc_coorddiff


def _make_sc_hgather(N_ACC, HD, E_pad, EW0):
    EWP = E_pad // 16
    EW1 = EWP - EW0
    NBUF = 4
    n0 = EW0 // SUB // NBUF
    n1 = EW1 // SUB // NBUF
    EWMAX = max(EW0, EW1)

    @functools.partial(
        pl.kernel,
        out_type=[
            jax.ShapeDtypeStruct((E_pad, HD), F32),
            jax.ShapeDtypeStruct((E_pad, HD), F32),
        ],
        mesh=_sc_mesh(),
        scratch_types=[
            pltpu.VMEM((EWMAX,), jnp.int32),
            pltpu.VMEM((EWMAX,), jnp.int32),
            pltpu.VMEM((NBUF, SUB, HD), F32),
            pltpu.SemaphoreType.DMA,
            pltpu.SemaphoreType.DMA,
        ],
        compiler_params=_SC_PARAMS,
    )
    def sc_hgather(hid_hbm, src_hbm, dst_hbm, hs_out, hd_out,
                   srcv, dstv, bufs, sem_i, sem_o):
        cid = lax.axis_index("c")
        sid = lax.axis_index("s")
        base = sid * EWP + cid * EW0
        ngrp = jnp.where(cid == 0, n0, n1)

        @pl.when(cid == 0)
        def _():
            pltpu.sync_copy(src_hbm.at[pl.ds(sid * EWP, EW0)],
                            srcv.at[pl.ds(0, EW0)])
            pltpu.sync_copy(dst_hbm.at[pl.ds(sid * EWP, EW0)],
                            dstv.at[pl.ds(0, EW0)])

        @pl.when(cid == 1)
        def _():
            pltpu.sync_copy(src_hbm.at[pl.ds(sid * EWP + EW0, EW1)],
                            srcv.at[pl.ds(0, EW1)])
            pltpu.sync_copy(dst_hbm.at[pl.ds(sid * EWP + EW0, EW1)],
                            dstv.at[pl.ds(0, EW1)])

        for idxv, out_hbm in ((srcv, hs_out), (dstv, hd_out)):
            def gbody(g, carry):
                j0 = g * NBUF
                cps = [
                    pltpu.async_copy(
                        hid_hbm.at[idxv.at[pl.ds((j0 + b) * SUB, SUB)]],
                        bufs.at[b], sem_i)
                    for b in range(NBUF)
                ]
                for cp in cps:
                    cp.wait()
                ws = [
                    pltpu.async_copy(
                        bufs.at[b],
                        out_hbm.at[pl.ds(base + (j0 + b) * SUB, SUB)],
                        sem_o)
                    for b in range(NBUF)
                ]
                for w in ws:
                    w.wait()
                return carry
            lax.fori_loop(0, ngrp, gbody, 0)

    return sc_hgather


def _make_sc_scatter_rows(N_ACC, HD, E_pad, EW):
    nsub = EW // SUB
    zrows = N_ACC // 16

    @functools.partial(
        pl.kernel,
        out_type=[
            jax.ShapeDtypeStruct((2, N_ACC, HD), F32),
        ],
        mesh=_sc_mesh(),
        scratch_types=[
            pltpu.VMEM_SHARED((N_ACC, HD), F32),
            pltpu.VMEM((EW // SUB, SUB), jnp.int32),
            pltpu.VMEM((2, SUB, HD), F32),
            pltpu.SemaphoreType.DMA,
        ],
        compiler_params=_SC_PARAMS,
    )
    def sc_scatter_rows(mout_hbm, dst2d_hbm, zeros_hbm, acc_out,
                        shared, idx2d, mrow, sem):
        cid = lax.axis_index("c")
        sid = lax.axis_index("s")
        wid = sid * 2 + cid
        base = wid * EW

        pltpu.sync_copy(zeros_hbm.at[pl.ds(sid * zrows, zrows)],
                        shared.at[pl.ds(sid * zrows, zrows)])
        plsc.subcore_barrier()

        pltpu.sync_copy(dst2d_hbm.at[pl.ds(wid * nsub, nsub)], idx2d)

        def sbody(p, carry):
            j0 = p * 2
            cps = [
                pltpu.async_copy(
                    mout_hbm.at[pl.ds(base + (j0 + b) * SUB, SUB)],
                    mrow.at[b], sem)
                for b in range(2)
            ]
            for b in range(2):
                cps[b].wait()
                pltpu.sync_copy(mrow.at[b], shared.at[idx2d.at[j0 + b]],
                                add=True)
            return carry

        lax.fori_loop(0, nsub // 2, sbody, 0)
        plsc.subcore_barrier()

        pltpu.sync_copy(shared.at[pl.ds(sid * zrows, zrows)],
                        acc_out.at[cid, pl.ds(sid * zrows, zrows)])

    return sc_scatter_rows


def _make_sc_scatter_scalars(N_ACC, E_pad, EW):
    nvec = EW // 16

    @functools.partial(
        pl.kernel,
        out_type=[
            jax.ShapeDtypeStruct((NW * 4 * N_ACC,), F32),
        ],
        mesh=_sc_mesh(),
        scratch_types=[
            pltpu.VMEM((EW,), jnp.int32),
            pltpu.VMEM((EW,), F32),
            pltpu.VMEM((EW,), F32),
            pltpu.VMEM((EW,), F32),
            pltpu.VMEM((N_ACC,), F32),
            pltpu.VMEM((N_ACC,), F32),
            pltpu.VMEM((N_ACC,), F32),
            pltpu.VMEM((N_ACC,), F32),
        ],
        compiler_params=_SC_PARAMS,
    )
    def sc_scatter_scalars(dst_hbm, tx_hbm, ty_hbm, tz_hbm, sacc_out,
                           dstv, txv, tyv, tzv, ax, ay, az, ad):
        cid = lax.axis_index("c")
        sid = lax.axis_index("s")
        wid = sid * 2 + cid
        base = wid * EW

        z16 = jnp.zeros((16,), F32)

        def zbody(i, carry):
            off = i * 16
            ax[pl.ds(off, 16)] = z16
            ay[pl.ds(off, 16)] = z16
            az[pl.ds(off, 16)] = z16
            ad[pl.ds(off, 16)] = z16
            return carry

        lax.fori_loop(0, N_ACC // 16, zbody, 0)

        pltpu.sync_copy(dst_hbm.at[pl.ds(base, EW)], dstv)
        pltpu.sync_copy(tx_hbm.at[pl.ds(base, EW)], txv)
        pltpu.sync_copy(ty_hbm.at[pl.ds(base, EW)], tyv)
        pltpu.sync_copy(tz_hbm.at[pl.ds(base, EW)], tzv)

        one16 = jnp.ones((16,), F32)

        def vbody(k, carry):
            off = k * 16
            d16 = dstv[pl.ds(off, 16)]
            plsc.addupdate_scatter(ax, [d16], txv[pl.ds(off, 16)])
            plsc.addupdate_scatter(ay, [d16], tyv[pl.ds(off, 16)])
            plsc.addupdate_scatter(az, [d16], tzv[pl.ds(off, 16)])
            plsc.addupdate_scatter(ad, [d16], one16)
            return carry

        lax.fori_loop(0, nvec, vbody, 0)

        for c, buf in ((0, ax), (1, ay), (2, az), (3, ad)):
            pltpu.sync_copy(
                buf, sacc_out.at[pl.ds((wid * 4 + c) * N_ACC, N_ACC)])

    return sc_scatter_scalars


def _edge_block_kernel(hs, hd, dx, dy, dz,
                       w1i, w1j, w0, b1, w2, b2, wc1, bc1, wc2,
                       mout, tx, ty, tz):
    dxr = dx[0]
    dyr = dy[0]
    dzr = dz[0]
    l2 = jnp.sqrt(dxr * dxr + dyr * dyr + dzr * dzr + 1e-12)
    pre1 = (jnp.dot(hs[...], w1i[...])
            + jnp.dot(hd[...], w1j[...])
            + lax.dot_general(l2, w0[...], (((0,), (0,)), ((), ())))
            + b1[...])
    t1 = jnp.tanh(pre1)
    mij = jnp.dot(t1, w2[...]) + b2[...]
    a1 = jnp.tanh(jnp.dot(mij, wc1[...]) + bc1[...])
    att = jnp.tanh(lax.dot_general(wc2[...], a1, (((0,), (1,)), ((), ()))))
    mout[...] = mij
    tx[0] = dxr * att
    ty[0] = dyr * att
    tz[0] = dzr * att


def _node_block_kernel(hid, a0, a1, wh1a, wh1b, bh1, wh2, bh2, hout):
    mi = a0[...] + a1[...]
    pre = (jnp.dot(hid[...], wh1a[...]) + jnp.dot(mi, wh1b[...]) + bh1[...])
    hout[...] = hid[...] + jnp.dot(jnp.tanh(pre), wh2[...]) + bh2[...]


def _coords_kernel(sacc, ct, cout):
    s = jnp.sum(sacc[...], axis=0)
    deg = jnp.maximum(s[3:4, :], 1.0)
    cout[...] = ct[...] + s[0:3, :] / deg


def kernel(coords, hidden, edges, W_m1, b_m1, W_m2, b_m2,
           W_c1, b_c1, W_c2, W_h1, b_h1, W_h2, b_h2):
    N, HD = hidden.shape
    E = edges.shape[1]
    MD = W_m2.shape[0]

    quantum = NW * SUB * BE // math.gcd(NW * SUB, BE)
    E_pad = -(-E // quantum) * quantum
    EW = E_pad // NW
    N_ACC = -(-(N + 1) // 128) * 128

    src = edges[0]
    dst = edges[1]
    pad = E_pad - E
    src_p = jnp.concatenate([src, jnp.zeros((pad,), jnp.int32)])
    dst_p = jnp.concatenate([dst, jnp.full((pad,), N, jnp.int32)])
    dst2d = dst_p.reshape(E_pad // SUB, SUB)

    hid_pad = jnp.pad(hidden, ((0, N_ACC - N), (0, 0)))
    cpad = jnp.pad(coords, ((0, N_ACC - N), (0, 1)))
    cflat = cpad.reshape(N_ACC * 4)
    zeros2d = jnp.zeros((N_ACC, HD), F32)

    dx, dy, dz = _make_sc_coorddiff(N_ACC, E_pad, EW)(cflat, src_p, dst_p)
    EW0 = (E_pad // 16) * 7 // 10 // 512 * 512
    hs, hd = _make_sc_hgather(N_ACC, HD, E_pad, EW0)(hid_pad, src_p, dst_p)

    NB = E_pad // BE
    dx3 = dx.reshape(NB, 1, BE)
    dy3 = dy.reshape(NB, 1, BE)
    dz3 = dz.reshape(NB, 1, BE)

    w0 = W_m1[0:1, :]
    w1i = W_m1[1:1 + HD, :]
    w1j = W_m1[1 + HD:1 + 2 * HD, :]
    b1 = b_m1.reshape(1, MD)
    b2 = b_m2.reshape(1, MD)
    bc1 = b_c1.reshape(1, MD)

    full = lambda shape: pl.BlockSpec(shape, lambda i: (0,) * len(shape))
    eb = pl.BlockSpec((BE, HD), lambda i: (i, 0))
    sb = pl.BlockSpec((1, 1, BE), lambda i: (i, 0, 0))

    mout, tx3, ty3, tz3 = pl.pallas_call(
        _edge_block_kernel,
        grid=(NB,),
        in_specs=[eb, eb, sb, sb, sb,
                  full((HD, MD)), full((HD, MD)), full((1, MD)),
                  full((1, MD)), full((MD, MD)), full((1, MD)),
                  full((MD, MD)), full((1, MD)), full((MD, 1))],
        out_specs=[eb, sb, sb, sb],
        out_shape=[
            jax.ShapeDtypeStruct((E_pad, MD), F32),
            jax.ShapeDtypeStruct((NB, 1, BE), F32),
            jax.ShapeDtypeStruct((NB, 1, BE), F32),
            jax.ShapeDtypeStruct((NB, 1, BE), F32),
        ],
    )(hs, hd, dx3, dy3, dz3, w1i, w1j, w0, b1, W_m2, b2, W_c1, bc1, W_c2)

    tx = tx3.reshape(E_pad)
    ty = ty3.reshape(E_pad)
    tz = tz3.reshape(E_pad)

    (acc2,) = _make_sc_scatter_rows(N_ACC, MD, E_pad, EW)(
        mout, dst2d, zeros2d)
    (sacc,) = _make_sc_scatter_scalars(N_ACC, E_pad, EW)(
        dst_p, tx, ty, tz)

    a0 = acc2[0, :N]
    a1 = acc2[1, :N]
    wh1a = W_h1[:HD, :]
    wh1b = W_h1[HD:, :]
    bh1 = b_h1.reshape(1, MD)
    bh2 = b_h2.reshape(1, HD)

    BN = 2000
    nb = pl.BlockSpec((BN, HD), lambda i: (i, 0))
    hidden_out = pl.pallas_call(
        _node_block_kernel,
        grid=(N // BN,),
        in_specs=[nb, nb, nb, full((HD, MD)), full((MD, MD)),
                  full((1, MD)), full((MD, HD)), full((1, HD))],
        out_specs=nb,
        out_shape=jax.ShapeDtypeStruct((N, HD), F32),
    )(hidden, a0, a1, wh1a, wh1b, bh1, W_h2, bh2)

    sacc3 = sacc.reshape(NW, 4, N_ACC)
    ct = jnp.pad(coords, ((0, N_ACC - N), (0, 0))).T
    cout = pl.pallas_call(
        _coords_kernel,
        grid=(1,),
        in_specs=[pl.BlockSpec((NW, 4, N_ACC), lambda i: (0, 0, 0)),
                  pl.BlockSpec((3, N_ACC), lambda i: (0, 0))],
        out_specs=pl.BlockSpec((3, N_ACC), lambda i: (0, 0)),
        out_shape=jax.ShapeDtypeStruct((3, N_ACC), F32),
    )(sacc3, ct)
    coords_out = cout.T[:N]

    return (coords_out, hidden_out)


# speedup vs baseline: 1.0398x; 1.0398x over previous
import functools
import math

import jax
import jax.numpy as jnp
from jax import lax
from jax.experimental import pallas as pl
from jax.experimental.pallas import tpu as pltpu
from jax.experimental.pallas import tpu_sc as plsc

F32 = jnp.float32
NW = 32
SUB = 128
BE = 1280


def _sc_mesh():
    return plsc.VectorSubcoreMesh(core_axis_name="c", subcore_axis_name="s")


_SC_PARAMS = pltpu.CompilerParams(needs_layout_passes=False)


def _make_sc_coorddiff(N_ACC, E_pad, EW):
    nsub_c = EW // 16

    @functools.partial(
        pl.kernel,
        out_type=[
            jax.ShapeDtypeStruct((E_pad,), F32),
            jax.ShapeDtypeStruct((E_pad,), F32),
            jax.ShapeDtypeStruct((E_pad,), F32),
        ],
        mesh=_sc_mesh(),
        scratch_types=[
            pltpu.VMEM((N_ACC * 4,), F32),
            pltpu.VMEM((EW,), jnp.int32),
            pltpu.VMEM((EW,), jnp.int32),
            pltpu.VMEM((EW,), F32),
            pltpu.VMEM((EW,), F32),
            pltpu.VMEM((EW,), F32),
        ],
        compiler_params=_SC_PARAMS,
    )
    def sc_coorddiff(cflat_hbm, src_hbm, dst_hbm,
                     dx_out, dy_out, dz_out,
                     cflat_v, srcv, dstv, dxb, dyb, dzb):
        cid = lax.axis_index("c")
        sid = lax.axis_index("s")
        wid = sid * 2 + cid
        base = wid * EW

        pltpu.sync_copy(cflat_hbm, cflat_v)
        pltpu.sync_copy(src_hbm.at[pl.ds(base, EW)], srcv)
        pltpu.sync_copy(dst_hbm.at[pl.ds(base, EW)], dstv)

        def cbody(k, carry):
            off = k * 16
            s16 = srcv[pl.ds(off, 16)] * 4
            d16 = dstv[pl.ds(off, 16)] * 4
            for c, buf in ((0, dxb), (1, dyb), (2, dzb)):
                a = plsc.load_gather(cflat_v, [s16 + c])
                b = plsc.load_gather(cflat_v, [d16 + c])
                buf[pl.ds(off, 16)] = a - b
            return carry

        lax.fori_loop(0, nsub_c, cbody, 0)
        pltpu.sync_copy(dxb, dx_out.at[pl.ds(base, EW)])
        pltpu.sync_copy(dyb, dy_out.at[pl.ds(base, EW)])
        pltpu.sync_copy(dzb, dz_out.at[pl.ds(base, EW)])

    return sc_coorddiff


def _make_sc_hgather(N_ACC, HD, E_pad, EW0):
    EWP = E_pad // 16
    EW1 = EWP - EW0
    NBUF = 4
    n0 = EW0 // SUB // NBUF
    n1 = EW1 // SUB // NBUF
    EWMAX = max(EW0, EW1)

    @functools.partial(
        pl.kernel,
        out_type=[
            jax.ShapeDtypeStruct((E_pad, HD), F32),
            jax.ShapeDtypeStruct((E_pad, HD), F32),
        ],
        mesh=_sc_mesh(),
        scratch_types=[
            pltpu.VMEM((EWMAX,), jnp.int32),
            pltpu.VMEM((EWMAX,), jnp.int32),
            pltpu.VMEM((NBUF, SUB, HD), F32),
            pltpu.SemaphoreType.DMA,
            pltpu.SemaphoreType.DMA,
        ],
        compiler_params=_SC_PARAMS,
    )
    def sc_hgather(hid_hbm, src_hbm, dst_hbm, hs_out, hd_out,
                   srcv, dstv, bufs, sem_i, sem_o):
        cid = lax.axis_index("c")
        sid = lax.axis_index("s")
        base = sid * EWP + cid * EW0
        ngrp = jnp.where(cid == 0, n0, n1)

        @pl.when(cid == 0)
        def _():
            pltpu.sync_copy(src_hbm.at[pl.ds(sid * EWP, EW0)],
                            srcv.at[pl.ds(0, EW0)])
            pltpu.sync_copy(dst_hbm.at[pl.ds(sid * EWP, EW0)],
                            dstv.at[pl.ds(0, EW0)])

        @pl.when(cid == 1)
        def _():
            pltpu.sync_copy(src_hbm.at[pl.ds(sid * EWP + EW0, EW1)],
                            srcv.at[pl.ds(0, EW1)])
            pltpu.sync_copy(dst_hbm.at[pl.ds(sid * EWP + EW0, EW1)],
                            dstv.at[pl.ds(0, EW1)])

        for idxv, out_hbm in ((srcv, hs_out), (dstv, hd_out)):
            def gbody(g, carry):
                j0 = g * NBUF
                cps = [
                    pltpu.async_copy(
                        hid_hbm.at[idxv.at[pl.ds((j0 + b) * SUB, SUB)]],
                        bufs.at[b], sem_i)
                    for b in range(NBUF)
                ]
                for cp in cps:
                    cp.wait()
                ws = [
                    pltpu.async_copy(
                        bufs.at[b],
                        out_hbm.at[pl.ds(base + (j0 + b) * SUB, SUB)],
                        sem_o)
                    for b in range(NBUF)
                ]
                for w in ws:
                    w.wait()
                return carry
            lax.fori_loop(0, ngrp, gbody, 0)

    return sc_hgather


def _make_sc_scatter_rows(N_ACC, HD, E_pad, EW):
    nsub = EW // SUB
    zrows = N_ACC // 16

    @functools.partial(
        pl.kernel,
        out_type=[
            jax.ShapeDtypeStruct((2, N_ACC, HD), F32),
        ],
        mesh=_sc_mesh(),
        scratch_types=[
            pltpu.VMEM_SHARED((N_ACC, HD), F32),
            pltpu.VMEM((EW // SUB, SUB), jnp.int32),
            pltpu.VMEM((2, SUB, HD), F32),
            pltpu.SemaphoreType.DMA,
        ],
        compiler_params=_SC_PARAMS,
    )
    def sc_scatter_rows(mout_hbm, dst2d_hbm, zeros_hbm, acc_out,
                        shared, idx2d, mrow, sem):
        cid = lax.axis_index("c")
        sid = lax.axis_index("s")
        wid = sid * 2 + cid
        base = wid * EW

        pltpu.sync_copy(zeros_hbm.at[pl.ds(sid * zrows, zrows)],
                        shared.at[pl.ds(sid * zrows, zrows)])
        plsc.subcore_barrier()

        pltpu.sync_copy(dst2d_hbm.at[pl.ds(wid * nsub, nsub)], idx2d)

        def sbody(p, carry):
            j0 = p * 2
            cps = [
                pltpu.async_copy(
                    mout_hbm.at[pl.ds(base + (j0 + b) * SUB, SUB)],
                    mrow.at[b], sem)
                for b in range(2)
            ]
            for b in range(2):
                cps[b].wait()
                pltpu.sync_copy(mrow.at[b], shared.at[idx2d.at[j0 + b]],
                                add=True)
            return carry

        lax.fori_loop(0, nsub // 2, sbody, 0)
        plsc.subcore_barrier()

        pltpu.sync_copy(shared.at[pl.ds(sid * zrows, zrows)],
                        acc_out.at[cid, pl.ds(sid * zrows, zrows)])

    return sc_scatter_rows


def _make_sc_scatter_scalars(N_ACC, E_pad, EW):
    nvec = EW // 16

    @functools.partial(
        pl.kernel,
        out_type=[
            jax.ShapeDtypeStruct((NW * 4 * N_ACC,), F32),
        ],
        mesh=_sc_mesh(),
        scratch_types=[
            pltpu.VMEM((EW,), jnp.int32),
            pltpu.VMEM((EW,), F32),
            pltpu.VMEM((EW,), F32),
            pltpu.VMEM((EW,), F32),
            pltpu.VMEM((N_ACC,), F32),
            pltpu.VMEM((N_ACC,), F32),
            pltpu.VMEM((N_ACC,), F32),
            pltpu.VMEM((N_ACC,), F32),
        ],
        compiler_params=_SC_PARAMS,
    )
    def sc_scatter_scalars(dst_hbm, tx_hbm, ty_hbm, tz_hbm, sacc_out,
                           dstv, txv, tyv, tzv, ax, ay, az, ad):
        cid = lax.axis_index("c")
        sid = lax.axis_index("s")
        wid = sid * 2 + cid
        base = wid * EW

        z16 = jnp.zeros((16,), F32)

        def zbody(i, carry):
            off = i * 16
            ax[pl.ds(off, 16)] = z16
            ay[pl.ds(off, 16)] = z16
            az[pl.ds(off, 16)] = z16
            ad[pl.ds(off, 16)] = z16
            return carry

        lax.fori_loop(0, N_ACC // 16, zbody, 0)

        pltpu.sync_copy(dst_hbm.at[pl.ds(base, EW)], dstv)
        pltpu.sync_copy(tx_hbm.at[pl.ds(base, EW)], txv)
        pltpu.sync_copy(ty_hbm.at[pl.ds(base, EW)], tyv)
        pltpu.sync_copy(tz_hbm.at[pl.ds(base, EW)], tzv)

        one16 = jnp.ones((16,), F32)

        def vbody(k, carry):
            off = k * 16
            d16 = dstv[pl.ds(off, 16)]
            plsc.addupdate_scatter(ax, [d16], txv[pl.ds(off, 16)])
            plsc.addupdate_scatter(ay, [d16], tyv[pl.ds(off, 16)])
            plsc.addupdate_scatter(az, [d16], tzv[pl.ds(off, 16)])
            plsc.addupdate_scatter(ad, [d16], one16)
            return carry

        lax.fori_loop(0, nvec, vbody, 0)

        for c, buf in ((0, ax), (1, ay), (2, az), (3, ad)):
            pltpu.sync_copy(
                buf, sacc_out.at[pl.ds((wid * 4 + c) * N_ACC, N_ACC)])

    return sc_scatter_scalars


def _edge_block_kernel(hs, hd, dx, dy, dz,
                       w1i, w1j, w0, b1, w2, b2, wc1, bc1, wc2,
                       mout, tx, ty, tz):
    dxr = dx[0]
    dyr = dy[0]
    dzr = dz[0]
    l2 = jnp.sqrt(dxr * dxr + dyr * dyr + dzr * dzr + 1e-12)
    pre1 = (jnp.dot(hs[...], w1i[...])
            + jnp.dot(hd[...], w1j[...])
            + lax.dot_general(l2, w0[...], (((0,), (0,)), ((), ())))
            + b1[...])
    t1 = jnp.tanh(pre1)
    mij = jnp.dot(t1, w2[...]) + b2[...]
    a1 = jnp.tanh(jnp.dot(mij, wc1[...]) + bc1[...])
    att = jnp.tanh(lax.dot_general(wc2[...], a1, (((0,), (1,)), ((), ()))))
    mout[...] = mij
    tx[0] = dxr * att
    ty[0] = dyr * att
    tz[0] = dzr * att


def _node_block_kernel(hid, a0, a1, wh1a, wh1b, bh1, wh2, bh2, hout):
    mi = a0[...] + a1[...]
    pre = (jnp.dot(hid[...], wh1a[...]) + jnp.dot(mi, wh1b[...]) + bh1[...])
    hout[...] = hid[...] + jnp.dot(jnp.tanh(pre), wh2[...]) + bh2[...]


def _coords_kernel(sacc, ct, cout):
    s = jnp.sum(sacc[...], axis=0)
    deg = jnp.maximum(s[3:4, :], 1.0)
    cout[...] = ct[...] + s[0:3, :] / deg


def kernel(coords, hidden, edges, W_m1, b_m1, W_m2, b_m2,
           W_c1, b_c1, W_c2, W_h1, b_h1, W_h2, b_h2):
    N, HD = hidden.shape
    E = edges.shape[1]
    MD = W_m2.shape[0]

    quantum = NW * SUB * BE // math.gcd(NW * SUB, BE)
    E_pad = -(-E // quantum) * quantum
    EW = E_pad // NW
    N_ACC = -(-(N + 1) // 128) * 128

    src = edges[0]
    dst = edges[1]
    pad = E_pad - E
    src_p = jnp.concatenate([src, jnp.zeros((pad,), jnp.int32)])
    dst_p = jnp.concatenate([dst, jnp.full((pad,), N, jnp.int32)])
    dst2d = dst_p.reshape(E_pad // SUB, SUB)

    hid_pad = jnp.pad(hidden, ((0, N_ACC - N), (0, 0)))
    cpad = jnp.pad(coords, ((0, N_ACC - N), (0, 1)))
    cflat = cpad.reshape(N_ACC * 4)
    zeros2d = jnp.zeros((N_ACC, HD), F32)

    dx, dy, dz = _make_sc_coorddiff(N_ACC, E_pad, EW)(cflat, src_p, dst_p)
    EW0 = (E_pad // 16) * 3 // 4 // 512 * 512
    hs, hd = _make_sc_hgather(N_ACC, HD, E_pad, EW0)(hid_pad, src_p, dst_p)

    NB = E_pad // BE
    dx3 = dx.reshape(NB, 1, BE)
    dy3 = dy.reshape(NB, 1, BE)
    dz3 = dz.reshape(NB, 1, BE)

    w0 = W_m1[0:1, :]
    w1i = W_m1[1:1 + HD, :]
    w1j = W_m1[1 + HD:1 + 2 * HD, :]
    b1 = b_m1.reshape(1, MD)
    b2 = b_m2.reshape(1, MD)
    bc1 = b_c1.reshape(1, MD)

    full = lambda shape: pl.BlockSpec(shape, lambda i: (0,) * len(shape))
    eb = pl.BlockSpec((BE, HD), lambda i: (i, 0))
    sb = pl.BlockSpec((1, 1, BE), lambda i: (i, 0, 0))

    mout, tx3, ty3, tz3 = pl.pallas_call(
        _edge_block_kernel,
        grid=(NB,),
        in_specs=[eb, eb, sb, sb, sb,
                  full((HD, MD)), full((HD, MD)), full((1, MD)),
                  full((1, MD)), full((MD, MD)), full((1, MD)),
                  full((MD, MD)), full((1, MD)), full((MD, 1))],
        out_specs=[eb, sb, sb, sb],
        out_shape=[
            jax.ShapeDtypeStruct((E_pad, MD), F32),
            jax.ShapeDtypeStruct((NB, 1, BE), F32),
            jax.ShapeDtypeStruct((NB, 1, BE), F32),
            jax.ShapeDtypeStruct((NB, 1, BE), F32),
        ],
    )(hs, hd, dx3, dy3, dz3, w1i, w1j, w0, b1, W_m2, b2, W_c1, bc1, W_c2)

    tx = tx3.reshape(E_pad)
    ty = ty3.reshape(E_pad)
    tz = tz3.reshape(E_pad)

    (acc2,) = _make_sc_scatter_rows(N_ACC, MD, E_pad, EW)(
        mout, dst2d, zeros2d)
    (sacc,) = _make_sc_scatter_scalars(N_ACC, E_pad, EW)(
        dst_p, tx, ty, tz)

    a0 = acc2[0, :N]
    a1 = acc2[1, :N]
    wh1a = W_h1[:HD, :]
    wh1b = W_h1[HD:, :]
    bh1 = b_h1.reshape(1, MD)
    bh2 = b_h2.reshape(1, HD)

    BN = 2000
    nb = pl.BlockSpec((BN, HD), lambda i: (i, 0))
    hidden_out = pl.pallas_call(
        _node_block_kernel,
        grid=(N // BN,),
        in_specs=[nb, nb, nb, full((HD, MD)), full((MD, MD)),
                  full((1, MD)), full((MD, HD)), full((1, HD))],
        out_specs=nb,
        out_shape=jax.ShapeDtypeStruct((N, HD), F32),
    )(hidden, a0, a1, wh1a, wh1b, bh1, W_h2, bh2)

    sacc3 = sacc.reshape(NW, 4, N_ACC)
    ct = jnp.pad(coords, ((0, N_ACC - N), (0, 0))).T
    cout = pl.pallas_call(
        _coords_kernel,
        grid=(1,),
        in_specs=[pl.BlockSpec((NW, 4, N_ACC), lambda i: (0, 0, 0)),
                  pl.BlockSpec((3, N_ACC), lambda i: (0, 0))],
        out_specs=pl.BlockSpec((3, N_ACC), lambda i: (0, 0)),
        out_shape=jax.ShapeDtypeStruct((3, N_ACC), F32),
    )(sacc3, ct)
    coords_out = cout.T[:N]

    return (coords_out, hidden_out)


# speedup vs baseline: 1.1063x; 1.0640x over previous
import functools
import math

import jax
import jax.numpy as jnp
from jax import lax
from jax.experimental import pallas as pl
from jax.experimental.pallas import tpu as pltpu
from jax.experimental.pallas import tpu_sc as plsc

F32 = jnp.float32
NW = 32
SUB = 128
BE = 1280


def _sc_mesh():
    return plsc.VectorSubcoreMesh(core_axis_name="c", subcore_axis_name="s")


_SC_PARAMS = pltpu.CompilerParams(needs_layout_passes=False)


def _make_sc_coorddiff(N_ACC, E_pad, EW):
    nsub_c = EW // 16

    @functools.partial(
        pl.kernel,
        out_type=[
            jax.ShapeDtypeStruct((E_pad,), F32),
            jax.ShapeDtypeStruct((E_pad,), F32),
            jax.ShapeDtypeStruct((E_pad,), F32),
        ],
        mesh=_sc_mesh(),
        scratch_types=[
            pltpu.VMEM((N_ACC * 4,), F32),
            pltpu.VMEM((EW,), jnp.int32),
            pltpu.VMEM((EW,), jnp.int32),
            pltpu.VMEM((EW,), F32),
            pltpu.VMEM((EW,), F32),
            pltpu.VMEM((EW,), F32),
        ],
        compiler_params=_SC_PARAMS,
    )
    def sc_coorddiff(cflat_hbm, src_hbm, dst_hbm,
                     dx_out, dy_out, dz_out,
                     cflat_v, srcv, dstv, dxb, dyb, dzb):
        cid = lax.axis_index("c")
        sid = lax.axis_index("s")
        wid = sid * 2 + cid
        base = wid * EW

        pltpu.sync_copy(cflat_hbm, cflat_v)
        pltpu.sync_copy(src_hbm.at[pl.ds(base, EW)], srcv)
        pltpu.sync_copy(dst_hbm.at[pl.ds(base, EW)], dstv)

        def cbody(k, carry):
            off = k * 16
            s16 = srcv[pl.ds(off, 16)] * 4
            d16 = dstv[pl.ds(off, 16)] * 4
            for c, buf in ((0, dxb), (1, dyb), (2, dzb)):
                a = plsc.load_gather(cflat_v, [s16 + c])
                b = plsc.load_gather(cflat_v, [d16 + c])
                buf[pl.ds(off, 16)] = a - b
            return carry

        lax.fori_loop(0, nsub_c, cbody, 0)
        pltpu.sync_copy(dxb, dx_out.at[pl.ds(base, EW)])
        pltpu.sync_copy(dyb, dy_out.at[pl.ds(base, EW)])
        pltpu.sync_copy(dzb, dz_out.at[pl.ds(base, EW)])

    return sc_coorddiff


def _make_sc_hgather(N_ACC, HP, E_pad, EW0):
    EWP = E_pad // 16
    EW1 = EWP - EW0
    NBUF = 8
    n0 = EW0 // SUB // NBUF
    n1 = EW1 // SUB // NBUF
    EWMAX = max(EW0, EW1)

    @functools.partial(
        pl.kernel,
        out_type=[
            jax.ShapeDtypeStruct((E_pad, HP), jnp.uint32),
            jax.ShapeDtypeStruct((E_pad, HP), jnp.uint32),
        ],
        mesh=_sc_mesh(),
        scratch_types=[
            pltpu.VMEM((EWMAX,), jnp.int32),
            pltpu.VMEM((EWMAX,), jnp.int32),
            pltpu.VMEM((NBUF, SUB, HP), jnp.uint32),
            pltpu.SemaphoreType.DMA,
            pltpu.SemaphoreType.DMA,
        ],
        compiler_params=pltpu.CompilerParams(
            needs_layout_passes=False, use_tc_tiling_on_sc=False),
    )
    def sc_hgather(hid_hbm, src_hbm, dst_hbm, hs_out, hd_out,
                   srcv, dstv, bufs, sem_i, sem_o):
        cid = lax.axis_index("c")
        sid = lax.axis_index("s")
        base = sid * EWP + cid * EW0
        ngrp = jnp.where(cid == 0, n0, n1)

        @pl.when(cid == 0)
        def _():
            pltpu.sync_copy(src_hbm.at[pl.ds(sid * EWP, EW0)],
                            srcv.at[pl.ds(0, EW0)])
            pltpu.sync_copy(dst_hbm.at[pl.ds(sid * EWP, EW0)],
                            dstv.at[pl.ds(0, EW0)])

        @pl.when(cid == 1)
        def _():
            pltpu.sync_copy(src_hbm.at[pl.ds(sid * EWP + EW0, EW1)],
                            srcv.at[pl.ds(0, EW1)])
            pltpu.sync_copy(dst_hbm.at[pl.ds(sid * EWP + EW0, EW1)],
                            dstv.at[pl.ds(0, EW1)])

        for idxv, out_hbm in ((srcv, hs_out), (dstv, hd_out)):
            def gbody(g, carry):
                j0 = g * NBUF
                cps = [
                    pltpu.async_copy(
                        hid_hbm.at[idxv.at[pl.ds((j0 + b) * SUB, SUB)]],
                        bufs.at[b], sem_i)
                    for b in range(NBUF)
                ]
                for cp in cps:
                    cp.wait()
                ws = [
                    pltpu.async_copy(
                        bufs.at[b],
                        out_hbm.at[pl.ds(base + (j0 + b) * SUB, SUB)],
                        sem_o)
                    for b in range(NBUF)
                ]
                for w in ws:
                    w.wait()
                return carry
            lax.fori_loop(0, ngrp, gbody, 0)

    return sc_hgather


def _make_sc_scatter_rows(N_ACC, HD, E_pad, EW):
    nsub = EW // SUB
    zrows = N_ACC // 16

    @functools.partial(
        pl.kernel,
        out_type=[
            jax.ShapeDtypeStruct((2, N_ACC, HD), F32),
        ],
        mesh=_sc_mesh(),
        scratch_types=[
            pltpu.VMEM_SHARED((N_ACC, HD), F32),
            pltpu.VMEM((EW // SUB, SUB), jnp.int32),
            pltpu.VMEM((2, SUB, HD), F32),
            pltpu.SemaphoreType.DMA,
        ],
        compiler_params=_SC_PARAMS,
    )
    def sc_scatter_rows(mout_hbm, dst2d_hbm, zeros_hbm, acc_out,
                        shared, idx2d, mrow, sem):
        cid = lax.axis_index("c")
        sid = lax.axis_index("s")
        wid = sid * 2 + cid
        base = wid * EW

        pltpu.sync_copy(zeros_hbm.at[pl.ds(sid * zrows, zrows)],
                        shared.at[pl.ds(sid * zrows, zrows)])
        plsc.subcore_barrier()

        pltpu.sync_copy(dst2d_hbm.at[pl.ds(wid * nsub, nsub)], idx2d)

        def sbody(p, carry):
            j0 = p * 2
            cps = [
                pltpu.async_copy(
                    mout_hbm.at[pl.ds(base + (j0 + b) * SUB, SUB)],
                    mrow.at[b], sem)
                for b in range(2)
            ]
            for b in range(2):
                cps[b].wait()
                pltpu.sync_copy(mrow.at[b], shared.at[idx2d.at[j0 + b]],
                                add=True)
            return carry

        lax.fori_loop(0, nsub // 2, sbody, 0)
        plsc.subcore_barrier()

        pltpu.sync_copy(shared.at[pl.ds(sid * zrows, zrows)],
                        acc_out.at[cid, pl.ds(sid * zrows, zrows)])

    return sc_scatter_rows


def _make_sc_scatter_scalars(N_ACC, E_pad, EW):
    nvec = EW // 16

    @functools.partial(
        pl.kernel,
        out_type=[
            jax.ShapeDtypeStruct((NW * 4 * N_ACC,), F32),
        ],
        mesh=_sc_mesh(),
        scratch_types=[
            pltpu.VMEM((EW,), jnp.int32),
            pltpu.VMEM((EW,), F32),
            pltpu.VMEM((EW,), F32),
            pltpu.VMEM((EW,), F32),
            pltpu.VMEM((N_ACC,), F32),
            pltpu.VMEM((N_ACC,), F32),
            pltpu.VMEM((N_ACC,), F32),
            pltpu.VMEM((N_ACC,), F32),
        ],
        compiler_params=_SC_PARAMS,
    )
    def sc_scatter_scalars(dst_hbm, tx_hbm, ty_hbm, tz_hbm, sacc_out,
                           dstv, txv, tyv, tzv, ax, ay, az, ad):
        cid = lax.axis_index("c")
        sid = lax.axis_index("s")
        wid = sid * 2 + cid
        base = wid * EW

        z16 = jnp.zeros((16,), F32)

        def zbody(i, carry):
            off = i * 16
            ax[pl.ds(off, 16)] = z16
            ay[pl.ds(off, 16)] = z16
            az[pl.ds(off, 16)] = z16
            ad[pl.ds(off, 16)] = z16
            return carry

        lax.fori_loop(0, N_ACC // 16, zbody, 0)

        pltpu.sync_copy(dst_hbm.at[pl.ds(base, EW)], dstv)
        pltpu.sync_copy(tx_hbm.at[pl.ds(base, EW)], txv)
        pltpu.sync_copy(ty_hbm.at[pl.ds(base, EW)], tyv)
        pltpu.sync_copy(tz_hbm.at[pl.ds(base, EW)], tzv)

        one16 = jnp.ones((16,), F32)

        def vbody(k, carry):
            off = k * 16
            d16 = dstv[pl.ds(off, 16)]
            plsc.addupdate_scatter(ax, [d16], txv[pl.ds(off, 16)])
            plsc.addupdate_scatter(ay, [d16], tyv[pl.ds(off, 16)])
            plsc.addupdate_scatter(az, [d16], tzv[pl.ds(off, 16)])
            plsc.addupdate_scatter(ad, [d16], one16)
            return carry

        lax.fori_loop(0, nvec, vbody, 0)

        for c, buf in ((0, ax), (1, ay), (2, az), (3, ad)):
            pltpu.sync_copy(
                buf, sacc_out.at[pl.ds((wid * 4 + c) * N_ACC, N_ACC)])

    return sc_scatter_scalars


def _edge_block_kernel(hs, hd, dx, dy, dz,
                       w1ie, w1io, w1je, w1jo, w0, b1, w2, b2, wc1, bc1, wc2,
                       mout, tx, ty, tz):
    dxr = dx[0]
    dyr = dy[0]
    dzr = dz[0]
    l2 = jnp.sqrt(dxr * dxr + dyr * dyr + dzr * dzr + 1e-12)
    us = hs[...]
    ud = hd[...]
    hi16 = jnp.uint32(0xFFFF0000)
    hse = lax.bitcast_convert_type(us << 16, F32)
    hso = lax.bitcast_convert_type(us & hi16, F32)
    hde = lax.bitcast_convert_type(ud << 16, F32)
    hdo = lax.bitcast_convert_type(ud & hi16, F32)
    pre1 = (jnp.dot(hse, w1ie[...])
            + jnp.dot(hso, w1io[...])
            + jnp.dot(hde, w1je[...])
            + jnp.dot(hdo, w1jo[...])
            + lax.dot_general(l2, w0[...], (((0,), (0,)), ((), ())))
            + b1[...])
    t1 = jnp.tanh(pre1)
    mij = jnp.dot(t1, w2[...]) + b2[...]
    a1 = jnp.tanh(jnp.dot(mij, wc1[...]) + bc1[...])
    att = jnp.tanh(lax.dot_general(wc2[...], a1, (((0,), (1,)), ((), ()))))
    mout[...] = mij
    tx[0] = dxr * att
    ty[0] = dyr * att
    tz[0] = dzr * att


def _node_block_kernel(hid, a0, a1, wh1a, wh1b, bh1, wh2, bh2, hout):
    mi = a0[...] + a1[...]
    pre = (jnp.dot(hid[...], wh1a[...]) + jnp.dot(mi, wh1b[...]) + bh1[...])
    hout[...] = hid[...] + jnp.dot(jnp.tanh(pre), wh2[...]) + bh2[...]


def _coords_kernel(sacc, ct, cout):
    s = jnp.sum(sacc[...], axis=0)
    deg = jnp.maximum(s[3:4, :], 1.0)
    cout[...] = ct[...] + s[0:3, :] / deg


def kernel(coords, hidden, edges, W_m1, b_m1, W_m2, b_m2,
           W_c1, b_c1, W_c2, W_h1, b_h1, W_h2, b_h2):
    N, HD = hidden.shape
    E = edges.shape[1]
    MD = W_m2.shape[0]

    quantum = NW * SUB * BE // math.gcd(NW * SUB, BE)
    E_pad = -(-E // quantum) * quantum
    EW = E_pad // NW
    N_ACC = -(-(N + 1) // 128) * 128

    src = edges[0]
    dst = edges[1]
    pad = E_pad - E
    src_p = jnp.concatenate([src, jnp.zeros((pad,), jnp.int32)])
    dst_p = jnp.concatenate([dst, jnp.full((pad,), N, jnp.int32)])
    dst2d = dst_p.reshape(E_pad // SUB, SUB)

    HP = HD // 2
    hu = lax.bitcast_convert_type(hidden.astype(jnp.bfloat16), jnp.uint16)
    hpk = (hu[:, 0::2].astype(jnp.uint32)
           | (hu[:, 1::2].astype(jnp.uint32) << 16))
    hid_pad = jnp.pad(hpk, ((0, N_ACC - N), (0, 0)))
    cpad = jnp.pad(coords, ((0, N_ACC - N), (0, 1)))
    cflat = cpad.reshape(N_ACC * 4)
    zeros2d = jnp.zeros((N_ACC, HD), F32)

    dx, dy, dz = _make_sc_coorddiff(N_ACC, E_pad, EW)(cflat, src_p, dst_p)
    EW0 = (E_pad // 16) * 3 // 4 // 1024 * 1024
    hs, hd = _make_sc_hgather(N_ACC, HP, E_pad, EW0)(hid_pad, src_p, dst_p)

    NB = E_pad // BE
    dx3 = dx.reshape(NB, 1, BE)
    dy3 = dy.reshape(NB, 1, BE)
    dz3 = dz.reshape(NB, 1, BE)

    w0 = W_m1[0:1, :]
    w1i = W_m1[1:1 + HD, :]
    w1j = W_m1[1 + HD:1 + 2 * HD, :]
    b1 = b_m1.reshape(1, MD)
    b2 = b_m2.reshape(1, MD)
    bc1 = b_c1.reshape(1, MD)

    full = lambda shape: pl.BlockSpec(shape, lambda i: (0,) * len(shape))
    eb = pl.BlockSpec((BE, HP), lambda i: (i, 0))
    sb = pl.BlockSpec((1, 1, BE), lambda i: (i, 0, 0))

    mout, tx3, ty3, tz3 = pl.pallas_call(
        _edge_block_kernel,
        grid=(NB,),
        in_specs=[eb, eb, sb, sb, sb,
                  full((HP, MD)), full((HP, MD)), full((HP, MD)),
                  full((HP, MD)), full((1, MD)),
                  full((1, MD)), full((MD, MD)), full((1, MD)),
                  full((MD, MD)), full((1, MD)), full((MD, 1))],
        out_specs=[pl.BlockSpec((BE, MD), lambda i: (i, 0)), sb, sb, sb],
        out_shape=[
            jax.ShapeDtypeStruct((E_pad, MD), F32),
            jax.ShapeDtypeStruct((NB, 1, BE), F32),
            jax.ShapeDtypeStruct((NB, 1, BE), F32),
            jax.ShapeDtypeStruct((NB, 1, BE), F32),
        ],
    )(hs, hd, dx3, dy3, dz3,
      w1i[0::2, :], w1i[1::2, :], w1j[0::2, :], w1j[1::2, :],
      w0, b1, W_m2, b2, W_c1, bc1, W_c2)

    tx = tx3.reshape(E_pad)
    ty = ty3.reshape(E_pad)
    tz = tz3.reshape(E_pad)

    (acc2,) = _make_sc_scatter_rows(N_ACC, MD, E_pad, EW)(
        mout, dst2d, zeros2d)
    (sacc,) = _make_sc_scatter_scalars(N_ACC, E_pad, EW)(
        dst_p, tx, ty, tz)

    a0 = acc2[0, :N]
    a1 = acc2[1, :N]
    wh1a = W_h1[:HD, :]
    wh1b = W_h1[HD:, :]
    bh1 = b_h1.reshape(1, MD)
    bh2 = b_h2.reshape(1, HD)

    BN = 2000
    nb = pl.BlockSpec((BN, HD), lambda i: (i, 0))
    hidden_out = pl.pallas_call(
        _node_block_kernel,
        grid=(N // BN,),
        in_specs=[nb, nb, nb, full((HD, MD)), full((MD, MD)),
                  full((1, MD)), full((MD, HD)), full((1, HD))],
        out_specs=nb,
        out_shape=jax.ShapeDtypeStruct((N, HD), F32),
    )(hidden, a0, a1, wh1a, wh1b, bh1, W_h2, bh2)

    sacc3 = sacc.reshape(NW, 4, N_ACC)
    ct = jnp.pad(coords, ((0, N_ACC - N), (0, 0))).T
    cout = pl.pallas_call(
        _coords_kernel,
        grid=(1,),
        in_specs=[pl.BlockSpec((NW, 4, N_ACC), lambda i: (0, 0, 0)),
                  pl.BlockSpec((3, N_ACC), lambda i: (0, 0))],
        out_specs=pl.BlockSpec((3, N_ACC), lambda i: (0, 0)),
        out_shape=jax.ShapeDtypeStruct((3, N_ACC), F32),
    )(sacc3, ct)
    coords_out = cout.T[:N]

    return (coords_out, hidden_out)


# speedup vs baseline: 1.1071x; 1.0007x over previous
import functools
import math

import jax
import jax.numpy as jnp
from jax import lax
from jax.experimental import pallas as pl
from jax.experimental.pallas import tpu as pltpu
from jax.experimental.pallas import tpu_sc as plsc

F32 = jnp.float32
NW = 32
SUB = 128
BE = 1280


def _sc_mesh():
    return plsc.VectorSubcoreMesh(core_axis_name="c", subcore_axis_name="s")


_SC_PARAMS = pltpu.CompilerParams(needs_layout_passes=False)


def _make_sc_coorddiff(N_ACC, E_pad, EW):
    nsub_c = EW // 16

    @functools.partial(
        pl.kernel,
        out_type=[
            jax.ShapeDtypeStruct((E_pad,), F32),
            jax.ShapeDtypeStruct((E_pad,), F32),
            jax.ShapeDtypeStruct((E_pad,), F32),
        ],
        mesh=_sc_mesh(),
        scratch_types=[
            pltpu.VMEM((N_ACC * 4,), F32),
            pltpu.VMEM((EW,), jnp.int32),
            pltpu.VMEM((EW,), jnp.int32),
            pltpu.VMEM((EW,), F32),
            pltpu.VMEM((EW,), F32),
            pltpu.VMEM((EW,), F32),
        ],
        compiler_params=_SC_PARAMS,
    )
    def sc_coorddiff(cflat_hbm, src_hbm, dst_hbm,
                     dx_out, dy_out, dz_out,
                     cflat_v, srcv, dstv, dxb, dyb, dzb):
        cid = lax.axis_index("c")
        sid = lax.axis_index("s")
        wid = sid * 2 + cid
        base = wid * EW

        pltpu.sync_copy(cflat_hbm, cflat_v)
        pltpu.sync_copy(src_hbm.at[pl.ds(base, EW)], srcv)
        pltpu.sync_copy(dst_hbm.at[pl.ds(base, EW)], dstv)

        def cbody(k, carry):
            off = k * 16
            s16 = srcv[pl.ds(off, 16)] * 4
            d16 = dstv[pl.ds(off, 16)] * 4
            for c, buf in ((0, dxb), (1, dyb), (2, dzb)):
                a = plsc.load_gather(cflat_v, [s16 + c])
                b = plsc.load_gather(cflat_v, [d16 + c])
                buf[pl.ds(off, 16)] = a - b
            return carry

        lax.fori_loop(0, nsub_c, cbody, 0)
        pltpu.sync_copy(dxb, dx_out.at[pl.ds(base, EW)])
        pltpu.sync_copy(dyb, dy_out.at[pl.ds(base, EW)])
        pltpu.sync_copy(dzb, dz_out.at[pl.ds(base, EW)])

    return sc_coorddiff


def _make_sc_hgather(N_ACC, HP, E_pad, EW0):
    EWP = E_pad // 16
    EW1 = EWP - EW0
    NBUF = 8
    n0 = EW0 // SUB // NBUF
    n1 = EW1 // SUB // NBUF
    EWMAX = max(EW0, EW1)

    @functools.partial(
        pl.kernel,
        out_type=[
            jax.ShapeDtypeStruct((E_pad, HP), jnp.uint32),
            jax.ShapeDtypeStruct((E_pad, HP), jnp.uint32),
        ],
        mesh=_sc_mesh(),
        scratch_types=[
            pltpu.VMEM((EWMAX,), jnp.int32),
            pltpu.VMEM((EWMAX,), jnp.int32),
            pltpu.VMEM((NBUF, SUB, HP), jnp.uint32),
            pltpu.SemaphoreType.DMA,
            pltpu.SemaphoreType.DMA,
        ],
        compiler_params=pltpu.CompilerParams(
            needs_layout_passes=False, use_tc_tiling_on_sc=False),
    )
    def sc_hgather(hid_hbm, src_hbm, dst_hbm, hs_out, hd_out,
                   srcv, dstv, bufs, sem_i, sem_o):
        cid = lax.axis_index("c")
        sid = lax.axis_index("s")
        base = sid * EWP + cid * EW0
        ngrp = jnp.where(cid == 0, n0, n1)

        @pl.when(cid == 0)
        def _():
            pltpu.sync_copy(src_hbm.at[pl.ds(sid * EWP, EW0)],
                            srcv.at[pl.ds(0, EW0)])
            pltpu.sync_copy(dst_hbm.at[pl.ds(sid * EWP, EW0)],
                            dstv.at[pl.ds(0, EW0)])

        @pl.when(cid == 1)
        def _():
            pltpu.sync_copy(src_hbm.at[pl.ds(sid * EWP + EW0, EW1)],
                            srcv.at[pl.ds(0, EW1)])
            pltpu.sync_copy(dst_hbm.at[pl.ds(sid * EWP + EW0, EW1)],
                            dstv.at[pl.ds(0, EW1)])

        for idxv, out_hbm in ((srcv, hs_out), (dstv, hd_out)):
            def gbody(g, carry):
                j0 = g * NBUF
                cps = [
                    pltpu.async_copy(
                        hid_hbm.at[idxv.at[pl.ds((j0 + b) * SUB, SUB)]],
                        bufs.at[b], sem_i)
                    for b in range(NBUF)
                ]
                for cp in cps:
                    cp.wait()
                ws = [
                    pltpu.async_copy(
                        bufs.at[b],
                        out_hbm.at[pl.ds(base + (j0 + b) * SUB, SUB)],
                        sem_o)
                    for b in range(NBUF)
                ]
                for w in ws:
                    w.wait()
                return carry
            lax.fori_loop(0, ngrp, gbody, 0)

    return sc_hgather


def _make_sc_scatter_rows(N_ACC, HD, E_pad, EW):
    nsub = EW // SUB
    zrows = N_ACC // 16

    @functools.partial(
        pl.kernel,
        out_type=[
            jax.ShapeDtypeStruct((2, N_ACC, HD), F32),
        ],
        mesh=_sc_mesh(),
        scratch_types=[
            pltpu.VMEM_SHARED((N_ACC, HD), F32),
            pltpu.VMEM((EW // SUB, SUB), jnp.int32),
            pltpu.VMEM((2, SUB, HD), F32),
            pltpu.SemaphoreType.DMA,
        ],
        compiler_params=_SC_PARAMS,
    )
    def sc_scatter_rows(mout_hbm, dst2d_hbm, zeros_hbm, acc_out,
                        shared, idx2d, mrow, sem):
        cid = lax.axis_index("c")
        sid = lax.axis_index("s")
        wid = sid * 2 + cid
        base = wid * EW

        pltpu.sync_copy(zeros_hbm.at[pl.ds(sid * zrows, zrows)],
                        shared.at[pl.ds(sid * zrows, zrows)])
        plsc.subcore_barrier()

        pltpu.sync_copy(dst2d_hbm.at[pl.ds(wid * nsub, nsub)], idx2d)

        def sbody(p, carry):
            j0 = p * 2
            cps = [
                pltpu.async_copy(
                    mout_hbm.at[pl.ds(base + (j0 + b) * SUB, SUB)],
                    mrow.at[b], sem)
                for b in range(2)
            ]
            for b in range(2):
                cps[b].wait()
                pltpu.sync_copy(mrow.at[b], shared.at[idx2d.at[j0 + b]],
                                add=True)
            return carry

        lax.fori_loop(0, nsub // 2, sbody, 0)
        plsc.subcore_barrier()

        pltpu.sync_copy(shared.at[pl.ds(sid * zrows, zrows)],
                        acc_out.at[cid, pl.ds(sid * zrows, zrows)])

    return sc_scatter_rows


def _make_sc_scatter_scalars(N_ACC, E_pad, EW):
    nvec = EW // 16

    @functools.partial(
        pl.kernel,
        out_type=[
            jax.ShapeDtypeStruct((NW * 4 * N_ACC,), F32),
        ],
        mesh=_sc_mesh(),
        scratch_types=[
            pltpu.VMEM((EW,), jnp.int32),
            pltpu.VMEM((EW,), F32),
            pltpu.VMEM((EW,), F32),
            pltpu.VMEM((EW,), F32),
            pltpu.VMEM((N_ACC,), F32),
            pltpu.VMEM((N_ACC,), F32),
            pltpu.VMEM((N_ACC,), F32),
            pltpu.VMEM((N_ACC,), F32),
        ],
        compiler_params=_SC_PARAMS,
    )
    def sc_scatter_scalars(dst_hbm, tx_hbm, ty_hbm, tz_hbm, sacc_out,
                           dstv, txv, tyv, tzv, ax, ay, az, ad):
        cid = lax.axis_index("c")
        sid = lax.axis_index("s")
        wid = sid * 2 + cid
        base = wid * EW

        z16 = jnp.zeros((16,), F32)

        def zbody(i, carry):
            off = i * 16
            ax[pl.ds(off, 16)] = z16
            ay[pl.ds(off, 16)] = z16
            az[pl.ds(off, 16)] = z16
            ad[pl.ds(off, 16)] = z16
            return carry

        lax.fori_loop(0, N_ACC // 16, zbody, 0)

        pltpu.sync_copy(dst_hbm.at[pl.ds(base, EW)], dstv)
        pltpu.sync_copy(tx_hbm.at[pl.ds(base, EW)], txv)
        pltpu.sync_copy(ty_hbm.at[pl.ds(base, EW)], tyv)
        pltpu.sync_copy(tz_hbm.at[pl.ds(base, EW)], tzv)

        one16 = jnp.ones((16,), F32)

        def vbody(k, carry):
            off = k * 16
            d16 = dstv[pl.ds(off, 16)]
            plsc.addupdate_scatter(ax, [d16], txv[pl.ds(off, 16)])
            plsc.addupdate_scatter(ay, [d16], tyv[pl.ds(off, 16)])
            plsc.addupdate_scatter(az, [d16], tzv[pl.ds(off, 16)])
            plsc.addupdate_scatter(ad, [d16], one16)
            return carry

        lax.fori_loop(0, nvec, vbody, 0)

        for c, buf in ((0, ax), (1, ay), (2, az), (3, ad)):
            pltpu.sync_copy(
                buf, sacc_out.at[pl.ds((wid * 4 + c) * N_ACC, N_ACC)])

    return sc_scatter_scalars


def _edge_block_kernel(hs, hd, dx, dy, dz,
                       w1ie, w1io, w1je, w1jo, w0, b1, w2, b2, wc1, bc1, wc2,
                       mout, tx, ty, tz):
    dxr = dx[0]
    dyr = dy[0]
    dzr = dz[0]
    l2 = jnp.sqrt(dxr * dxr + dyr * dyr + dzr * dzr + 1e-12)
    us = hs[...]
    ud = hd[...]
    hi16 = jnp.uint32(0xFFFF0000)
    BF = jnp.bfloat16
    hse = lax.bitcast_convert_type(us << 16, F32).astype(BF)
    hso = lax.bitcast_convert_type(us & hi16, F32).astype(BF)
    hde = lax.bitcast_convert_type(ud << 16, F32).astype(BF)
    hdo = lax.bitcast_convert_type(ud & hi16, F32).astype(BF)
    dot = functools.partial(jnp.dot, preferred_element_type=F32)
    pre1 = (dot(hse, w1ie[...].astype(BF))
            + dot(hso, w1io[...].astype(BF))
            + dot(hde, w1je[...].astype(BF))
            + dot(hdo, w1jo[...].astype(BF))
            + lax.dot_general(l2, w0[...], (((0,), (0,)), ((), ())))
            + b1[...])
    t1 = jnp.tanh(pre1).astype(BF)
    mij = dot(t1, w2[...].astype(BF)) + b2[...]
    a1 = jnp.tanh(dot(mij.astype(BF), wc1[...].astype(BF)) + bc1[...])
    att = jnp.tanh(lax.dot_general(
        wc2[...].astype(BF), a1.astype(BF), (((0,), (1,)), ((), ())),
        preferred_element_type=F32))
    mout[...] = mij
    tx[0] = dxr * att
    ty[0] = dyr * att
    tz[0] = dzr * att


def _node_block_kernel(hid, a0, a1, wh1a, wh1b, bh1, wh2, bh2, hout):
    mi = a0[...] + a1[...]
    pre = (jnp.dot(hid[...], wh1a[...]) + jnp.dot(mi, wh1b[...]) + bh1[...])
    hout[...] = hid[...] + jnp.dot(jnp.tanh(pre), wh2[...]) + bh2[...]


def _coords_kernel(sacc, ct, cout):
    s = jnp.sum(sacc[...], axis=0)
    deg = jnp.maximum(s[3:4, :], 1.0)
    cout[...] = ct[...] + s[0:3, :] / deg


def kernel(coords, hidden, edges, W_m1, b_m1, W_m2, b_m2,
           W_c1, b_c1, W_c2, W_h1, b_h1, W_h2, b_h2):
    N, HD = hidden.shape
    E = edges.shape[1]
    MD = W_m2.shape[0]

    quantum = NW * SUB * BE // math.gcd(NW * SUB, BE)
    E_pad = -(-E // quantum) * quantum
    EW = E_pad // NW
    N_ACC = -(-(N + 1) // 128) * 128

    src = edges[0]
    dst = edges[1]
    pad = E_pad - E
    src_p = jnp.concatenate([src, jnp.zeros((pad,), jnp.int32)])
    dst_p = jnp.concatenate([dst, jnp.full((pad,), N, jnp.int32)])
    dst2d = dst_p.reshape(E_pad // SUB, SUB)

    HP = HD // 2
    hu = lax.bitcast_convert_type(hidden.astype(jnp.bfloat16), jnp.uint16)
    hpk = (hu[:, 0::2].astype(jnp.uint32)
           | (hu[:, 1::2].astype(jnp.uint32) << 16))
    hid_pad = jnp.pad(hpk, ((0, N_ACC - N), (0, 0)))
    cpad = jnp.pad(coords, ((0, N_ACC - N), (0, 1)))
    cflat = cpad.reshape(N_ACC * 4)
    zeros2d = jnp.zeros((N_ACC, HD), F32)

    dx, dy, dz = _make_sc_coorddiff(N_ACC, E_pad, EW)(cflat, src_p, dst_p)
    EW0 = (E_pad // 16) * 3 // 4 // 1024 * 1024
    hs, hd = _make_sc_hgather(N_ACC, HP, E_pad, EW0)(hid_pad, src_p, dst_p)

    NB = E_pad // BE
    dx3 = dx.reshape(NB, 1, BE)
    dy3 = dy.reshape(NB, 1, BE)
    dz3 = dz.reshape(NB, 1, BE)

    w0 = W_m1[0:1, :]
    w1i = W_m1[1:1 + HD, :]
    w1j = W_m1[1 + HD:1 + 2 * HD, :]
    b1 = b_m1.reshape(1, MD)
    b2 = b_m2.reshape(1, MD)
    bc1 = b_c1.reshape(1, MD)

    full = lambda shape: pl.BlockSpec(shape, lambda i: (0,) * len(shape))
    eb = pl.BlockSpec((BE, HP), lambda i: (i, 0))
    sb = pl.BlockSpec((1, 1, BE), lambda i: (i, 0, 0))

    mout, tx3, ty3, tz3 = pl.pallas_call(
        _edge_block_kernel,
        grid=(NB,),
        in_specs=[eb, eb, sb, sb, sb,
                  full((HP, MD)), full((HP, MD)), full((HP, MD)),
                  full((HP, MD)), full((1, MD)),
                  full((1, MD)), full((MD, MD)), full((1, MD)),
                  full((MD, MD)), full((1, MD)), full((MD, 1))],
        out_specs=[pl.BlockSpec((BE, MD), lambda i: (i, 0)), sb, sb, sb],
        out_shape=[
            jax.ShapeDtypeStruct((E_pad, MD), F32),
            jax.ShapeDtypeStruct((NB, 1, BE), F32),
            jax.ShapeDtypeStruct((NB, 1, BE), F32),
            jax.ShapeDtypeStruct((NB, 1, BE), F32),
        ],
    )(hs, hd, dx3, dy3, dz3,
      w1i[0::2, :], w1i[1::2, :], w1j[0::2, :], w1j[1::2, :],
      w0, b1, W_m2, b2, W_c1, bc1, W_c2)

    tx = tx3.reshape(E_pad)
    ty = ty3.reshape(E_pad)
    tz = tz3.reshape(E_pad)

    (acc2,) = _make_sc_scatter_rows(N_ACC, MD, E_pad, EW)(
        mout, dst2d, zeros2d)
    (sacc,) = _make_sc_scatter_scalars(N_ACC, E_pad, EW)(
        dst_p, tx, ty, tz)

    a0 = acc2[0, :N]
    a1 = acc2[1, :N]
    wh1a = W_h1[:HD, :]
    wh1b = W_h1[HD:, :]
    bh1 = b_h1.reshape(1, MD)
    bh2 = b_h2.reshape(1, HD)

    BN = 2000
    nb = pl.BlockSpec((BN, HD), lambda i: (i, 0))
    hidden_out = pl.pallas_call(
        _node_block_kernel,
        grid=(N // BN,),
        in_specs=[nb, nb, nb, full((HD, MD)), full((MD, MD)),
                  full((1, MD)), full((MD, HD)), full((1, HD))],
        out_specs=nb,
        out_shape=jax.ShapeDtypeStruct((N, HD), F32),
    )(hidden, a0, a1, wh1a, wh1b, bh1, W_h2, bh2)

    sacc3 = sacc.reshape(NW, 4, N_ACC)
    ct = jnp.pad(coords, ((0, N_ACC - N), (0, 0))).T
    cout = pl.pallas_call(
        _coords_kernel,
        grid=(1,),
        in_specs=[pl.BlockSpec((NW, 4, N_ACC), lambda i: (0, 0, 0)),
                  pl.BlockSpec((3, N_ACC), lambda i: (0, 0))],
        out_specs=pl.BlockSpec((3, N_ACC), lambda i: (0, 0)),
        out_shape=jax.ShapeDtypeStruct((3, N_ACC), F32),
    )(sacc3, ct)
    coords_out = cout.T[:N]

    return (coords_out, hidden_out)


# speedup vs baseline: 1.1509x; 1.0396x over previous
import functools
import math

import jax
import jax.numpy as jnp
from jax import lax
from jax.experimental import pallas as pl
from jax.experimental.pallas import tpu as pltpu
from jax.experimental.pallas import tpu_sc as plsc

F32 = jnp.float32
NW = 32
SUB = 128
BE = 2560


def _sc_mesh():
    return plsc.VectorSubcoreMesh(core_axis_name="c", subcore_axis_name="s")


_SC_PARAMS = pltpu.CompilerParams(needs_layout_passes=False)


def _make_sc_coorddiff(N_ACC, E_pad, EW):
    nsub_c = EW // 16

    @functools.partial(
        pl.kernel,
        out_type=[
            jax.ShapeDtypeStruct((E_pad,), F32),
            jax.ShapeDtypeStruct((E_pad,), F32),
            jax.ShapeDtypeStruct((E_pad,), F32),
        ],
        mesh=_sc_mesh(),
        scratch_types=[
            pltpu.VMEM((N_ACC * 4,), F32),
            pltpu.VMEM((EW,), jnp.int32),
            pltpu.VMEM((EW,), jnp.int32),
            pltpu.VMEM((EW,), F32),
            pltpu.VMEM((EW,), F32),
            pltpu.VMEM((EW,), F32),
        ],
        compiler_params=_SC_PARAMS,
    )
    def sc_coorddiff(cflat_hbm, src_hbm, dst_hbm,
                     dx_out, dy_out, dz_out,
                     cflat_v, srcv, dstv, dxb, dyb, dzb):
        cid = lax.axis_index("c")
        sid = lax.axis_index("s")
        wid = sid * 2 + cid
        base = wid * EW

        pltpu.sync_copy(cflat_hbm, cflat_v)
        pltpu.sync_copy(src_hbm.at[pl.ds(base, EW)], srcv)
        pltpu.sync_copy(dst_hbm.at[pl.ds(base, EW)], dstv)

        def cbody(k, carry):
            off = k * 16
            s16 = srcv[pl.ds(off, 16)] * 4
            d16 = dstv[pl.ds(off, 16)] * 4
            for c, buf in ((0, dxb), (1, dyb), (2, dzb)):
                a = plsc.load_gather(cflat_v, [s16 + c])
                b = plsc.load_gather(cflat_v, [d16 + c])
                buf[pl.ds(off, 16)] = a - b
            return carry

        lax.fori_loop(0, nsub_c, cbody, 0)
        pltpu.sync_copy(dxb, dx_out.at[pl.ds(base, EW)])
        pltpu.sync_copy(dyb, dy_out.at[pl.ds(base, EW)])
        pltpu.sync_copy(dzb, dz_out.at[pl.ds(base, EW)])

    return sc_coorddiff


def _make_sc_hgather(N_ACC, HP, E_pad, EW0):
    EWP = E_pad // 16
    EW1 = EWP - EW0
    NBUF = 8
    n0 = EW0 // SUB // NBUF
    n1 = EW1 // SUB // NBUF
    EWMAX = max(EW0, EW1)

    @functools.partial(
        pl.kernel,
        out_type=[
            jax.ShapeDtypeStruct((E_pad, HP), jnp.uint32),
            jax.ShapeDtypeStruct((E_pad, HP), jnp.uint32),
        ],
        mesh=_sc_mesh(),
        scratch_types=[
            pltpu.VMEM((EWMAX,), jnp.int32),
            pltpu.VMEM((EWMAX,), jnp.int32),
            pltpu.VMEM((NBUF, SUB, HP), jnp.uint32),
            pltpu.SemaphoreType.DMA,
            pltpu.SemaphoreType.DMA,
        ],
        compiler_params=pltpu.CompilerParams(
            needs_layout_passes=False, use_tc_tiling_on_sc=False),
    )
    def sc_hgather(hid_hbm, src_hbm, dst_hbm, hs_out, hd_out,
                   srcv, dstv, bufs, sem_i, sem_o):
        cid = lax.axis_index("c")
        sid = lax.axis_index("s")
        base = sid * EWP + cid * EW0
        ngrp = jnp.where(cid == 0, n0, n1)

        @pl.when(cid == 0)
        def _():
            pltpu.sync_copy(src_hbm.at[pl.ds(sid * EWP, EW0)],
                            srcv.at[pl.ds(0, EW0)])
            pltpu.sync_copy(dst_hbm.at[pl.ds(sid * EWP, EW0)],
                            dstv.at[pl.ds(0, EW0)])

        @pl.when(cid == 1)
        def _():
            pltpu.sync_copy(src_hbm.at[pl.ds(sid * EWP + EW0, EW1)],
                            srcv.at[pl.ds(0, EW1)])
            pltpu.sync_copy(dst_hbm.at[pl.ds(sid * EWP + EW0, EW1)],
                            dstv.at[pl.ds(0, EW1)])

        for idxv, out_hbm in ((srcv, hs_out), (dstv, hd_out)):
            def gbody(g, carry):
                j0 = g * NBUF
                cps = [
                    pltpu.async_copy(
                        hid_hbm.at[idxv.at[pl.ds((j0 + b) * SUB, SUB)]],
                        bufs.at[b], sem_i)
                    for b in range(NBUF)
                ]
                for cp in cps:
                    cp.wait()
                ws = [
                    pltpu.async_copy(
                        bufs.at[b],
                        out_hbm.at[pl.ds(base + (j0 + b) * SUB, SUB)],
                        sem_o)
                    for b in range(NBUF)
                ]
                for w in ws:
                    w.wait()
                return carry
            lax.fori_loop(0, ngrp, gbody, 0)

    return sc_hgather


def _make_sc_scatter_rows(N_ACC, HD, E_pad, EW):
    nsub = EW // SUB
    zrows = N_ACC // 16

    @functools.partial(
        pl.kernel,
        out_type=[
            jax.ShapeDtypeStruct((2, N_ACC, HD), F32),
        ],
        mesh=_sc_mesh(),
        scratch_types=[
            pltpu.VMEM_SHARED((N_ACC, HD), F32),
            pltpu.VMEM((EW // SUB, SUB), jnp.int32),
            pltpu.VMEM((2, SUB, HD), F32),
            pltpu.SemaphoreType.DMA,
        ],
        compiler_params=_SC_PARAMS,
    )
    def sc_scatter_rows(mout_hbm, dst2d_hbm, zeros_hbm, acc_out,
                        shared, idx2d, mrow, sem):
        cid = lax.axis_index("c")
        sid = lax.axis_index("s")
        wid = sid * 2 + cid
        base = wid * EW

        pltpu.sync_copy(zeros_hbm.at[pl.ds(sid * zrows, zrows)],
                        shared.at[pl.ds(sid * zrows, zrows)])
        plsc.subcore_barrier()

        pltpu.sync_copy(dst2d_hbm.at[pl.ds(wid * nsub, nsub)], idx2d)

        def sbody(p, carry):
            j0 = p * 2
            cps = [
                pltpu.async_copy(
                    mout_hbm.at[pl.ds(base + (j0 + b) * SUB, SUB)],
                    mrow.at[b], sem)
                for b in range(2)
            ]
            for b in range(2):
                cps[b].wait()
                pltpu.sync_copy(mrow.at[b], shared.at[idx2d.at[j0 + b]],
                                add=True)
            return carry

        lax.fori_loop(0, nsub // 2, sbody, 0)
        plsc.subcore_barrier()

        pltpu.sync_copy(shared.at[pl.ds(sid * zrows, zrows)],
                        acc_out.at[cid, pl.ds(sid * zrows, zrows)])

    return sc_scatter_rows


def _make_sc_scatter_scalars(N_ACC, E_pad, EW):
    nvec = EW // 16

    @functools.partial(
        pl.kernel,
        out_type=[
            jax.ShapeDtypeStruct((NW * 4 * N_ACC,), F32),
        ],
        mesh=_sc_mesh(),
        scratch_types=[
            pltpu.VMEM((EW,), jnp.int32),
            pltpu.VMEM((EW,), F32),
            pltpu.VMEM((EW,), F32),
            pltpu.VMEM((EW,), F32),
            pltpu.VMEM((N_ACC,), F32),
            pltpu.VMEM((N_ACC,), F32),
            pltpu.VMEM((N_ACC,), F32),
            pltpu.VMEM((N_ACC,), F32),
        ],
        compiler_params=_SC_PARAMS,
    )
    def sc_scatter_scalars(dst_hbm, tx_hbm, ty_hbm, tz_hbm, sacc_out,
                           dstv, txv, tyv, tzv, ax, ay, az, ad):
        cid = lax.axis_index("c")
        sid = lax.axis_index("s")
        wid = sid * 2 + cid
        base = wid * EW

        z16 = jnp.zeros((16,), F32)

        def zbody(i, carry):
            off = i * 16
            ax[pl.ds(off, 16)] = z16
            ay[pl.ds(off, 16)] = z16
            az[pl.ds(off, 16)] = z16
            ad[pl.ds(off, 16)] = z16
            return carry

        lax.fori_loop(0, N_ACC // 16, zbody, 0)

        pltpu.sync_copy(dst_hbm.at[pl.ds(base, EW)], dstv)
        pltpu.sync_copy(tx_hbm.at[pl.ds(base, EW)], txv)
        pltpu.sync_copy(ty_hbm.at[pl.ds(base, EW)], tyv)
        pltpu.sync_copy(tz_hbm.at[pl.ds(base, EW)], tzv)

        one16 = jnp.ones((16,), F32)

        def vbody(k, carry):
            off = k * 16
            d16 = dstv[pl.ds(off, 16)]
            plsc.addupdate_scatter(ax, [d16], txv[pl.ds(off, 16)])
            plsc.addupdate_scatter(ay, [d16], tyv[pl.ds(off, 16)])
            plsc.addupdate_scatter(az, [d16], tzv[pl.ds(off, 16)])
            plsc.addupdate_scatter(ad, [d16], one16)
            return carry

        lax.fori_loop(0, nvec, vbody, 0)

        for c, buf in ((0, ax), (1, ay), (2, az), (3, ad)):
            pltpu.sync_copy(
                buf, sacc_out.at[pl.ds((wid * 4 + c) * N_ACC, N_ACC)])

    return sc_scatter_scalars


def _edge_block_kernel(hs, hd, dx, dy, dz,
                       w1ie, w1io, w1je, w1jo, w0, b1, w2, b2, wc1, bc1, wc2,
                       mout, tx, ty, tz):
    dxr = dx[0]
    dyr = dy[0]
    dzr = dz[0]
    l2 = jnp.sqrt(dxr * dxr + dyr * dyr + dzr * dzr + 1e-12)
    us = hs[...]
    ud = hd[...]
    hi16 = jnp.uint32(0xFFFF0000)
    BF = jnp.bfloat16
    hse = lax.bitcast_convert_type(us << 16, F32).astype(BF)
    hso = lax.bitcast_convert_type(us & hi16, F32).astype(BF)
    hde = lax.bitcast_convert_type(ud << 16, F32).astype(BF)
    hdo = lax.bitcast_convert_type(ud & hi16, F32).astype(BF)
    dot = functools.partial(jnp.dot, preferred_element_type=F32)
    pre1 = (dot(hse, w1ie[...].astype(BF))
            + dot(hso, w1io[...].astype(BF))
            + dot(hde, w1je[...].astype(BF))
            + dot(hdo, w1jo[...].astype(BF))
            + lax.dot_general(l2, w0[...], (((0,), (0,)), ((), ())))
            + b1[...])
    t1 = jnp.tanh(pre1).astype(BF)
    mij = dot(t1, w2[...].astype(BF)) + b2[...]
    a1 = jnp.tanh(dot(mij.astype(BF), wc1[...].astype(BF)) + bc1[...])
    att = jnp.tanh(lax.dot_general(
        wc2[...].astype(BF), a1.astype(BF), (((0,), (1,)), ((), ())),
        preferred_element_type=F32))
    mout[...] = mij
    tx[0] = dxr * att
    ty[0] = dyr * att
    tz[0] = dzr * att


def _node_block_kernel(hid, a0, a1, wh1a, wh1b, bh1, wh2, bh2, hout):
    mi = a0[...] + a1[...]
    pre = (jnp.dot(hid[...], wh1a[...]) + jnp.dot(mi, wh1b[...]) + bh1[...])
    hout[...] = hid[...] + jnp.dot(jnp.tanh(pre), wh2[...]) + bh2[...]


def _coords_kernel(sacc, ct, cout):
    s = jnp.sum(sacc[...], axis=0)
    deg = jnp.maximum(s[3:4, :], 1.0)
    cout[...] = ct[...] + s[0:3, :] / deg


def kernel(coords, hidden, edges, W_m1, b_m1, W_m2, b_m2,
           W_c1, b_c1, W_c2, W_h1, b_h1, W_h2, b_h2):
    N, HD = hidden.shape
    E = edges.shape[1]
    MD = W_m2.shape[0]

    quantum = NW * SUB * BE // math.gcd(NW * SUB, BE)
    E_pad = -(-E // quantum) * quantum
    EW = E_pad // NW
    N_ACC = -(-(N + 1) // 128) * 128

    src = edges[0]
    dst = edges[1]
    pad = E_pad - E
    src_p = jnp.concatenate([src, jnp.zeros((pad,), jnp.int32)])
    dst_p = jnp.concatenate([dst, jnp.full((pad,), N, jnp.int32)])
    dst2d = dst_p.reshape(E_pad // SUB, SUB)

    HP = HD // 2
    hu = lax.bitcast_convert_type(hidden.astype(jnp.bfloat16), jnp.uint16)
    hpk = (hu[:, 0::2].astype(jnp.uint32)
           | (hu[:, 1::2].astype(jnp.uint32) << 16))
    hid_pad = jnp.pad(hpk, ((0, N_ACC - N), (0, 0)))
    cpad = jnp.pad(coords, ((0, N_ACC - N), (0, 1)))
    cflat = cpad.reshape(N_ACC * 4)
    zeros2d = jnp.zeros((N_ACC, HD), F32)

    dx, dy, dz = _make_sc_coorddiff(N_ACC, E_pad, EW)(cflat, src_p, dst_p)
    EW0 = (E_pad // 16) * 3 // 4 // 1024 * 1024
    hs, hd = _make_sc_hgather(N_ACC, HP, E_pad, EW0)(hid_pad, src_p, dst_p)

    NB = E_pad // BE
    dx3 = dx.reshape(NB, 1, BE)
    dy3 = dy.reshape(NB, 1, BE)
    dz3 = dz.reshape(NB, 1, BE)

    w0 = W_m1[0:1, :]
    w1i = W_m1[1:1 + HD, :]
    w1j = W_m1[1 + HD:1 + 2 * HD, :]
    b1 = b_m1.reshape(1, MD)
    b2 = b_m2.reshape(1, MD)
    bc1 = b_c1.reshape(1, MD)

    full = lambda shape: pl.BlockSpec(shape, lambda i: (0,) * len(shape))
    eb = pl.BlockSpec((BE, HP), lambda i: (i, 0))
    sb = pl.BlockSpec((1, 1, BE), lambda i: (i, 0, 0))

    mout, tx3, ty3, tz3 = pl.pallas_call(
        _edge_block_kernel,
        grid=(NB,),
        in_specs=[eb, eb, sb, sb, sb,
                  full((HP, MD)), full((HP, MD)), full((HP, MD)),
                  full((HP, MD)), full((1, MD)),
                  full((1, MD)), full((MD, MD)), full((1, MD)),
                  full((MD, MD)), full((1, MD)), full((MD, 1))],
        out_specs=[pl.BlockSpec((BE, MD), lambda i: (i, 0)), sb, sb, sb],
        out_shape=[
            jax.ShapeDtypeStruct((E_pad, MD), F32),
            jax.ShapeDtypeStruct((NB, 1, BE), F32),
            jax.ShapeDtypeStruct((NB, 1, BE), F32),
            jax.ShapeDtypeStruct((NB, 1, BE), F32),
        ],
    )(hs, hd, dx3, dy3, dz3,
      w1i[0::2, :], w1i[1::2, :], w1j[0::2, :], w1j[1::2, :],
      w0, b1, W_m2, b2, W_c1, bc1, W_c2)

    tx = tx3.reshape(E_pad)
    ty = ty3.reshape(E_pad)
    tz = tz3.reshape(E_pad)

    (acc2,) = _make_sc_scatter_rows(N_ACC, MD, E_pad, EW)(
        mout, dst2d, zeros2d)
    (sacc,) = _make_sc_scatter_scalars(N_ACC, E_pad, EW)(
        dst_p, tx, ty, tz)

    a0 = acc2[0, :N]
    a1 = acc2[1, :N]
    wh1a = W_h1[:HD, :]
    wh1b = W_h1[HD:, :]
    bh1 = b_h1.reshape(1, MD)
    bh2 = b_h2.reshape(1, HD)

    BN = 2000
    nb = pl.BlockSpec((BN, HD), lambda i: (i, 0))
    hidden_out = pl.pallas_call(
        _node_block_kernel,
        grid=(N // BN,),
        in_specs=[nb, nb, nb, full((HD, MD)), full((MD, MD)),
                  full((1, MD)), full((MD, HD)), full((1, HD))],
        out_specs=nb,
        out_shape=jax.ShapeDtypeStruct((N, HD), F32),
    )(hidden, a0, a1, wh1a, wh1b, bh1, W_h2, bh2)

    sacc3 = sacc.reshape(NW, 4, N_ACC)
    ct = jnp.pad(coords, ((0, N_ACC - N), (0, 0))).T
    cout = pl.pallas_call(
        _coords_kernel,
        grid=(1,),
        in_specs=[pl.BlockSpec((NW, 4, N_ACC), lambda i: (0, 0, 0)),
                  pl.BlockSpec((3, N_ACC), lambda i: (0, 0))],
        out_specs=pl.BlockSpec((3, N_ACC), lambda i: (0, 0)),
        out_shape=jax.ShapeDtypeStruct((3, N_ACC), F32),
    )(sacc3, ct)
    coords_out = cout.T[:N]

    return (coords_out, hidden_out)


# speedup vs baseline: 1.1681x; 1.0150x over previous
import functools
import math

import jax
import jax.numpy as jnp
from jax import lax
from jax.experimental import pallas as pl
from jax.experimental.pallas import tpu as pltpu
from jax.experimental.pallas import tpu_sc as plsc

F32 = jnp.float32
NW = 32
SUB = 128
BE = 5120


def _sc_mesh():
    return plsc.VectorSubcoreMesh(core_axis_name="c", subcore_axis_name="s")


_SC_PARAMS = pltpu.CompilerParams(needs_layout_passes=False)


def _make_sc_coorddiff(N_ACC, E_pad, EW):
    nsub_c = EW // 16

    @functools.partial(
        pl.kernel,
        out_type=[
            jax.ShapeDtypeStruct((E_pad,), F32),
            jax.ShapeDtypeStruct((E_pad,), F32),
            jax.ShapeDtypeStruct((E_pad,), F32),
        ],
        mesh=_sc_mesh(),
        scratch_types=[
            pltpu.VMEM((N_ACC * 4,), F32),
            pltpu.VMEM((EW,), jnp.int32),
            pltpu.VMEM((EW,), jnp.int32),
            pltpu.VMEM((EW,), F32),
            pltpu.VMEM((EW,), F32),
            pltpu.VMEM((EW,), F32),
        ],
        compiler_params=_SC_PARAMS,
    )
    def sc_coorddiff(cflat_hbm, src_hbm, dst_hbm,
                     dx_out, dy_out, dz_out,
                     cflat_v, srcv, dstv, dxb, dyb, dzb):
        cid = lax.axis_index("c")
        sid = lax.axis_index("s")
        wid = sid * 2 + cid
        base = wid * EW

        pltpu.sync_copy(cflat_hbm, cflat_v)
        pltpu.sync_copy(src_hbm.at[pl.ds(base, EW)], srcv)
        pltpu.sync_copy(dst_hbm.at[pl.ds(base, EW)], dstv)

        def cbody(k, carry):
            off = k * 16
            s16 = srcv[pl.ds(off, 16)] * 4
            d16 = dstv[pl.ds(off, 16)] * 4
            for c, buf in ((0, dxb), (1, dyb), (2, dzb)):
                a = plsc.load_gather(cflat_v, [s16 + c])
                b = plsc.load_gather(cflat_v, [d16 + c])
                buf[pl.ds(off, 16)] = a - b
            return carry

        lax.fori_loop(0, nsub_c, cbody, 0)
        pltpu.sync_copy(dxb, dx_out.at[pl.ds(base, EW)])
        pltpu.sync_copy(dyb, dy_out.at[pl.ds(base, EW)])
        pltpu.sync_copy(dzb, dz_out.at[pl.ds(base, EW)])

    return sc_coorddiff


def _make_sc_hgather(N_ACC, HP, E_pad, EW0):
    EWP = E_pad // 16
    EW1 = EWP - EW0
    NBUF = 8
    n0 = EW0 // SUB // NBUF
    n1 = EW1 // SUB // NBUF
    EWMAX = max(EW0, EW1)

    @functools.partial(
        pl.kernel,
        out_type=[
            jax.ShapeDtypeStruct((E_pad, HP), jnp.uint32),
            jax.ShapeDtypeStruct((E_pad, HP), jnp.uint32),
        ],
        mesh=_sc_mesh(),
        scratch_types=[
            pltpu.VMEM((EWMAX,), jnp.int32),
            pltpu.VMEM((EWMAX,), jnp.int32),
            pltpu.VMEM((NBUF, SUB, HP), jnp.uint32),
            pltpu.SemaphoreType.DMA,
            pltpu.SemaphoreType.DMA,
        ],
        compiler_params=pltpu.CompilerParams(
            needs_layout_passes=False, use_tc_tiling_on_sc=False),
    )
    def sc_hgather(hid_hbm, src_hbm, dst_hbm, hs_out, hd_out,
                   srcv, dstv, bufs, sem_i, sem_o):
        cid = lax.axis_index("c")
        sid = lax.axis_index("s")
        base = sid * EWP + cid * EW0
        ngrp = jnp.where(cid == 0, n0, n1)

        @pl.when(cid == 0)
        def _():
            pltpu.sync_copy(src_hbm.at[pl.ds(sid * EWP, EW0)],
                            srcv.at[pl.ds(0, EW0)])
            pltpu.sync_copy(dst_hbm.at[pl.ds(sid * EWP, EW0)],
                            dstv.at[pl.ds(0, EW0)])

        @pl.when(cid == 1)
        def _():
            pltpu.sync_copy(src_hbm.at[pl.ds(sid * EWP + EW0, EW1)],
                            srcv.at[pl.ds(0, EW1)])
            pltpu.sync_copy(dst_hbm.at[pl.ds(sid * EWP + EW0, EW1)],
                            dstv.at[pl.ds(0, EW1)])

        for idxv, out_hbm in ((srcv, hs_out), (dstv, hd_out)):
            def gbody(g, carry):
                j0 = g * NBUF
                cps = [
                    pltpu.async_copy(
                        hid_hbm.at[idxv.at[pl.ds((j0 + b) * SUB, SUB)]],
                        bufs.at[b], sem_i)
                    for b in range(NBUF)
                ]
                for cp in cps:
                    cp.wait()
                ws = [
                    pltpu.async_copy(
                        bufs.at[b],
                        out_hbm.at[pl.ds(base + (j0 + b) * SUB, SUB)],
                        sem_o)
                    for b in range(NBUF)
                ]
                for w in ws:
                    w.wait()
                return carry
            lax.fori_loop(0, ngrp, gbody, 0)

    return sc_hgather


def _make_sc_scatter_rows(N_ACC, HD, E_pad, EW):
    nsub = EW // SUB
    zrows = N_ACC // 16

    @functools.partial(
        pl.kernel,
        out_type=[
            jax.ShapeDtypeStruct((2, N_ACC, HD), F32),
        ],
        mesh=_sc_mesh(),
        scratch_types=[
            pltpu.VMEM_SHARED((N_ACC, HD), F32),
            pltpu.VMEM((EW // SUB, SUB), jnp.int32),
            pltpu.VMEM((2, SUB, HD), F32),
            pltpu.SemaphoreType.DMA,
        ],
        compiler_params=_SC_PARAMS,
    )
    def sc_scatter_rows(mout_hbm, dst2d_hbm, zeros_hbm, acc_out,
                        shared, idx2d, mrow, sem):
        cid = lax.axis_index("c")
        sid = lax.axis_index("s")
        wid = sid * 2 + cid
        base = wid * EW

        pltpu.sync_copy(zeros_hbm.at[pl.ds(sid * zrows, zrows)],
                        shared.at[pl.ds(sid * zrows, zrows)])
        plsc.subcore_barrier()

        pltpu.sync_copy(dst2d_hbm.at[pl.ds(wid * nsub, nsub)], idx2d)

        def sbody(p, carry):
            j0 = p * 2
            cps = [
                pltpu.async_copy(
                    mout_hbm.at[pl.ds(base + (j0 + b) * SUB, SUB)],
                    mrow.at[b], sem)
                for b in range(2)
            ]
            for b in range(2):
                cps[b].wait()
                pltpu.sync_copy(mrow.at[b], shared.at[idx2d.at[j0 + b]],
                                add=True)
            return carry

        lax.fori_loop(0, nsub // 2, sbody, 0)
        plsc.subcore_barrier()

        pltpu.sync_copy(shared.at[pl.ds(sid * zrows, zrows)],
                        acc_out.at[cid, pl.ds(sid * zrows, zrows)])

    return sc_scatter_rows


def _make_sc_scatter_scalars(N_ACC, E_pad, EW):
    nvec = EW // 16

    @functools.partial(
        pl.kernel,
        out_type=[
            jax.ShapeDtypeStruct((NW * 4 * N_ACC,), F32),
        ],
        mesh=_sc_mesh(),
        scratch_types=[
            pltpu.VMEM((EW,), jnp.int32),
            pltpu.VMEM((EW,), F32),
            pltpu.VMEM((EW,), F32),
            pltpu.VMEM((EW,), F32),
            pltpu.VMEM((N_ACC,), F32),
            pltpu.VMEM((N_ACC,), F32),
            pltpu.VMEM((N_ACC,), F32),
            pltpu.VMEM((N_ACC,), F32),
        ],
        compiler_params=_SC_PARAMS,
    )
    def sc_scatter_scalars(dst_hbm, tx_hbm, ty_hbm, tz_hbm, sacc_out,
                           dstv, txv, tyv, tzv, ax, ay, az, ad):
        cid = lax.axis_index("c")
        sid = lax.axis_index("s")
        wid = sid * 2 + cid
        base = wid * EW

        z16 = jnp.zeros((16,), F32)

        def zbody(i, carry):
            off = i * 16
            ax[pl.ds(off, 16)] = z16
            ay[pl.ds(off, 16)] = z16
            az[pl.ds(off, 16)] = z16
            ad[pl.ds(off, 16)] = z16
            return carry

        lax.fori_loop(0, N_ACC // 16, zbody, 0)

        pltpu.sync_copy(dst_hbm.at[pl.ds(base, EW)], dstv)
        pltpu.sync_copy(tx_hbm.at[pl.ds(base, EW)], txv)
        pltpu.sync_copy(ty_hbm.at[pl.ds(base, EW)], tyv)
        pltpu.sync_copy(tz_hbm.at[pl.ds(base, EW)], tzv)

        one16 = jnp.ones((16,), F32)

        def vbody(k, carry):
            off = k * 16
            d16 = dstv[pl.ds(off, 16)]
            plsc.addupdate_scatter(ax, [d16], txv[pl.ds(off, 16)])
            plsc.addupdate_scatter(ay, [d16], tyv[pl.ds(off, 16)])
            plsc.addupdate_scatter(az, [d16], tzv[pl.ds(off, 16)])
            plsc.addupdate_scatter(ad, [d16], one16)
            return carry

        lax.fori_loop(0, nvec, vbody, 0)

        for c, buf in ((0, ax), (1, ay), (2, az), (3, ad)):
            pltpu.sync_copy(
                buf, sacc_out.at[pl.ds((wid * 4 + c) * N_ACC, N_ACC)])

    return sc_scatter_scalars


def _edge_block_kernel(hs, hd, dx, dy, dz,
                       w1ie, w1io, w1je, w1jo, w0, b1, w2, b2, wc1, bc1, wc2,
                       mout, tx, ty, tz):
    dxr = dx[0]
    dyr = dy[0]
    dzr = dz[0]
    l2 = jnp.sqrt(dxr * dxr + dyr * dyr + dzr * dzr + 1e-12)
    us = hs[...]
    ud = hd[...]
    hi16 = jnp.uint32(0xFFFF0000)
    BF = jnp.bfloat16
    hse = lax.bitcast_convert_type(us << 16, F32).astype(BF)
    hso = lax.bitcast_convert_type(us & hi16, F32).astype(BF)
    hde = lax.bitcast_convert_type(ud << 16, F32).astype(BF)
    hdo = lax.bitcast_convert_type(ud & hi16, F32).astype(BF)
    dot = functools.partial(jnp.dot, preferred_element_type=F32)
    pre1 = (dot(hse, w1ie[...].astype(BF))
            + dot(hso, w1io[...].astype(BF))
            + dot(hde, w1je[...].astype(BF))
            + dot(hdo, w1jo[...].astype(BF))
            + lax.dot_general(l2, w0[...], (((0,), (0,)), ((), ())))
            + b1[...])
    t1 = jnp.tanh(pre1).astype(BF)
    mij = dot(t1, w2[...].astype(BF)) + b2[...]
    a1 = jnp.tanh(dot(mij.astype(BF), wc1[...].astype(BF)) + bc1[...])
    att = jnp.tanh(lax.dot_general(
        wc2[...].astype(BF), a1.astype(BF), (((0,), (1,)), ((), ())),
        preferred_element_type=F32))
    mout[...] = mij
    tx[0] = dxr * att
    ty[0] = dyr * att
    tz[0] = dzr * att


def _node_block_kernel(hid, a0, a1, wh1a, wh1b, bh1, wh2, bh2, hout):
    mi = a0[...] + a1[...]
    pre = (jnp.dot(hid[...], wh1a[...]) + jnp.dot(mi, wh1b[...]) + bh1[...])
    hout[...] = hid[...] + jnp.dot(jnp.tanh(pre), wh2[...]) + bh2[...]


def _coords_kernel(sacc, ct, cout):
    s = jnp.sum(sacc[...], axis=0)
    deg = jnp.maximum(s[3:4, :], 1.0)
    cout[...] = ct[...] + s[0:3, :] / deg


def kernel(coords, hidden, edges, W_m1, b_m1, W_m2, b_m2,
           W_c1, b_c1, W_c2, W_h1, b_h1, W_h2, b_h2):
    N, HD = hidden.shape
    E = edges.shape[1]
    MD = W_m2.shape[0]

    quantum = NW * SUB * BE // math.gcd(NW * SUB, BE)
    E_pad = -(-E // quantum) * quantum
    EW = E_pad // NW
    N_ACC = -(-(N + 1) // 128) * 128

    src = edges[0]
    dst = edges[1]
    pad = E_pad - E
    src_p = jnp.concatenate([src, jnp.zeros((pad,), jnp.int32)])
    dst_p = jnp.concatenate([dst, jnp.full((pad,), N, jnp.int32)])
    dst2d = dst_p.reshape(E_pad // SUB, SUB)

    HP = HD // 2
    hu = lax.bitcast_convert_type(hidden.astype(jnp.bfloat16), jnp.uint16)
    hpk = (hu[:, 0::2].astype(jnp.uint32)
           | (hu[:, 1::2].astype(jnp.uint32) << 16))
    hid_pad = jnp.pad(hpk, ((0, N_ACC - N), (0, 0)))
    cpad = jnp.pad(coords, ((0, N_ACC - N), (0, 1)))
    cflat = cpad.reshape(N_ACC * 4)
    zeros2d = jnp.zeros((N_ACC, HD), F32)

    dx, dy, dz = _make_sc_coorddiff(N_ACC, E_pad, EW)(cflat, src_p, dst_p)
    EW0 = (E_pad // 16) * 3 // 4 // 1024 * 1024
    hs, hd = _make_sc_hgather(N_ACC, HP, E_pad, EW0)(hid_pad, src_p, dst_p)

    NB = E_pad // BE
    dx3 = dx.reshape(NB, 1, BE)
    dy3 = dy.reshape(NB, 1, BE)
    dz3 = dz.reshape(NB, 1, BE)

    w0 = W_m1[0:1, :]
    w1i = W_m1[1:1 + HD, :]
    w1j = W_m1[1 + HD:1 + 2 * HD, :]
    b1 = b_m1.reshape(1, MD)
    b2 = b_m2.reshape(1, MD)
    bc1 = b_c1.reshape(1, MD)

    full = lambda shape: pl.BlockSpec(shape, lambda i: (0,) * len(shape))
    eb = pl.BlockSpec((BE, HP), lambda i: (i, 0))
    sb = pl.BlockSpec((1, 1, BE), lambda i: (i, 0, 0))

    mout, tx3, ty3, tz3 = pl.pallas_call(
        _edge_block_kernel,
        grid=(NB,),
        in_specs=[eb, eb, sb, sb, sb,
                  full((HP, MD)), full((HP, MD)), full((HP, MD)),
                  full((HP, MD)), full((1, MD)),
                  full((1, MD)), full((MD, MD)), full((1, MD)),
                  full((MD, MD)), full((1, MD)), full((MD, 1))],
        out_specs=[pl.BlockSpec((BE, MD), lambda i: (i, 0)), sb, sb, sb],
        out_shape=[
            jax.ShapeDtypeStruct((E_pad, MD), F32),
            jax.ShapeDtypeStruct((NB, 1, BE), F32),
            jax.ShapeDtypeStruct((NB, 1, BE), F32),
            jax.ShapeDtypeStruct((NB, 1, BE), F32),
        ],
    )(hs, hd, dx3, dy3, dz3,
      w1i[0::2, :], w1i[1::2, :], w1j[0::2, :], w1j[1::2, :],
      w0, b1, W_m2, b2, W_c1, bc1, W_c2)

    tx = tx3.reshape(E_pad)
    ty = ty3.reshape(E_pad)
    tz = tz3.reshape(E_pad)

    (acc2,) = _make_sc_scatter_rows(N_ACC, MD, E_pad, EW)(
        mout, dst2d, zeros2d)
    (sacc,) = _make_sc_scatter_scalars(N_ACC, E_pad, EW)(
        dst_p, tx, ty, tz)

    a0 = acc2[0, :N]
    a1 = acc2[1, :N]
    wh1a = W_h1[:HD, :]
    wh1b = W_h1[HD:, :]
    bh1 = b_h1.reshape(1, MD)
    bh2 = b_h2.reshape(1, HD)

    BN = 2000
    nb = pl.BlockSpec((BN, HD), lambda i: (i, 0))
    hidden_out = pl.pallas_call(
        _node_block_kernel,
        grid=(N // BN,),
        in_specs=[nb, nb, nb, full((HD, MD)), full((MD, MD)),
                  full((1, MD)), full((MD, HD)), full((1, HD))],
        out_specs=nb,
        out_shape=jax.ShapeDtypeStruct((N, HD), F32),
    )(hidden, a0, a1, wh1a, wh1b, bh1, W_h2, bh2)

    sacc3 = sacc.reshape(NW, 4, N_ACC)
    ct = jnp.pad(coords, ((0, N_ACC - N), (0, 0))).T
    cout = pl.pallas_call(
        _coords_kernel,
        grid=(1,),
        in_specs=[pl.BlockSpec((NW, 4, N_ACC), lambda i: (0, 0, 0)),
                  pl.BlockSpec((3, N_ACC), lambda i: (0, 0))],
        out_specs=pl.BlockSpec((3, N_ACC), lambda i: (0, 0)),
        out_shape=jax.ShapeDtypeStruct((3, N_ACC), F32),
    )(sacc3, ct)
    coords_out = cout.T[:N]

    return (coords_out, hidden_out)


# speedup vs baseline: 1.1730x; 1.0041x over previous
import functools
import math

import jax
import jax.numpy as jnp
from jax import lax
from jax.experimental import pallas as pl
from jax.experimental.pallas import tpu as pltpu
from jax.experimental.pallas import tpu_sc as plsc

F32 = jnp.float32
NW = 32
SUB = 128
BE = 10240


def _sc_mesh():
    return plsc.VectorSubcoreMesh(core_axis_name="c", subcore_axis_name="s")


_SC_PARAMS = pltpu.CompilerParams(needs_layout_passes=False)


def _make_sc_coorddiff(N_ACC, E_pad, EW):
    nsub_c = EW // 16

    @functools.partial(
        pl.kernel,
        out_type=[
            jax.ShapeDtypeStruct((E_pad,), F32),
            jax.ShapeDtypeStruct((E_pad,), F32),
            jax.ShapeDtypeStruct((E_pad,), F32),
        ],
        mesh=_sc_mesh(),
        scratch_types=[
            pltpu.VMEM((N_ACC * 4,), F32),
            pltpu.VMEM((EW,), jnp.int32),
            pltpu.VMEM((EW,), jnp.int32),
            pltpu.VMEM((EW,), F32),
            pltpu.VMEM((EW,), F32),
            pltpu.VMEM((EW,), F32),
        ],
        compiler_params=_SC_PARAMS,
    )
    def sc_coorddiff(cflat_hbm, src_hbm, dst_hbm,
                     dx_out, dy_out, dz_out,
                     cflat_v, srcv, dstv, dxb, dyb, dzb):
        cid = lax.axis_index("c")
        sid = lax.axis_index("s")
        wid = sid * 2 + cid
        base = wid * EW

        pltpu.sync_copy(cflat_hbm, cflat_v)
        pltpu.sync_copy(src_hbm.at[pl.ds(base, EW)], srcv)
        pltpu.sync_copy(dst_hbm.at[pl.ds(base, EW)], dstv)

        def cbody(k, carry):
            off = k * 16
            s16 = srcv[pl.ds(off, 16)] * 4
            d16 = dstv[pl.ds(off, 16)] * 4
            for c, buf in ((0, dxb), (1, dyb), (2, dzb)):
                a = plsc.load_gather(cflat_v, [s16 + c])
                b = plsc.load_gather(cflat_v, [d16 + c])
                buf[pl.ds(off, 16)] = a - b
            return carry

        lax.fori_loop(0, nsub_c, cbody, 0)
        pltpu.sync_copy(dxb, dx_out.at[pl.ds(base, EW)])
        pltpu.sync_copy(dyb, dy_out.at[pl.ds(base, EW)])
        pltpu.sync_copy(dzb, dz_out.at[pl.ds(base, EW)])

    return sc_coorddiff


def _make_sc_hgather(N_ACC, HP, E_pad, EW0):
    EWP = E_pad // 16
    EW1 = EWP - EW0
    NBUF = 8
    n0 = EW0 // SUB // NBUF
    n1 = EW1 // SUB // NBUF
    EWMAX = max(EW0, EW1)

    @functools.partial(
        pl.kernel,
        out_type=[
            jax.ShapeDtypeStruct((E_pad, HP), jnp.uint32),
            jax.ShapeDtypeStruct((E_pad, HP), jnp.uint32),
        ],
        mesh=_sc_mesh(),
        scratch_types=[
            pltpu.VMEM((EWMAX,), jnp.int32),
            pltpu.VMEM((EWMAX,), jnp.int32),
            pltpu.VMEM((NBUF, SUB, HP), jnp.uint32),
            pltpu.SemaphoreType.DMA,
            pltpu.SemaphoreType.DMA,
        ],
        compiler_params=pltpu.CompilerParams(
            needs_layout_passes=False, use_tc_tiling_on_sc=False),
    )
    def sc_hgather(hid_hbm, src_hbm, dst_hbm, hs_out, hd_out,
                   srcv, dstv, bufs, sem_i, sem_o):
        cid = lax.axis_index("c")
        sid = lax.axis_index("s")
        base = sid * EWP + cid * EW0
        ngrp = jnp.where(cid == 0, n0, n1)

        @pl.when(cid == 0)
        def _():
            pltpu.sync_copy(src_hbm.at[pl.ds(sid * EWP, EW0)],
                            srcv.at[pl.ds(0, EW0)])
            pltpu.sync_copy(dst_hbm.at[pl.ds(sid * EWP, EW0)],
                            dstv.at[pl.ds(0, EW0)])

        @pl.when(cid == 1)
        def _():
            pltpu.sync_copy(src_hbm.at[pl.ds(sid * EWP + EW0, EW1)],
                            srcv.at[pl.ds(0, EW1)])
            pltpu.sync_copy(dst_hbm.at[pl.ds(sid * EWP + EW0, EW1)],
                            dstv.at[pl.ds(0, EW1)])

        for idxv, out_hbm in ((srcv, hs_out), (dstv, hd_out)):
            def gbody(g, carry):
                j0 = g * NBUF
                cps = [
                    pltpu.async_copy(
                        hid_hbm.at[idxv.at[pl.ds((j0 + b) * SUB, SUB)]],
                        bufs.at[b], sem_i)
                    for b in range(NBUF)
                ]
                for cp in cps:
                    cp.wait()
                ws = [
                    pltpu.async_copy(
                        bufs.at[b],
                        out_hbm.at[pl.ds(base + (j0 + b) * SUB, SUB)],
                        sem_o)
                    for b in range(NBUF)
                ]
                for w in ws:
                    w.wait()
                return carry
            lax.fori_loop(0, ngrp, gbody, 0)

    return sc_hgather


def _make_sc_scatter_rows(N_ACC, HD, E_pad, EW):
    nsub = EW // SUB
    zrows = N_ACC // 16

    @functools.partial(
        pl.kernel,
        out_type=[
            jax.ShapeDtypeStruct((2, N_ACC, HD), F32),
        ],
        mesh=_sc_mesh(),
        scratch_types=[
            pltpu.VMEM_SHARED((N_ACC, HD), F32),
            pltpu.VMEM((EW // SUB, SUB), jnp.int32),
            pltpu.VMEM((2, SUB, HD), F32),
            pltpu.SemaphoreType.DMA,
        ],
        compiler_params=_SC_PARAMS,
    )
    def sc_scatter_rows(mout_hbm, dst2d_hbm, zeros_hbm, acc_out,
                        shared, idx2d, mrow, sem):
        cid = lax.axis_index("c")
        sid = lax.axis_index("s")
        wid = sid * 2 + cid
        base = wid * EW

        pltpu.sync_copy(zeros_hbm.at[pl.ds(sid * zrows, zrows)],
                        shared.at[pl.ds(sid * zrows, zrows)])
        plsc.subcore_barrier()

        pltpu.sync_copy(dst2d_hbm.at[pl.ds(wid * nsub, nsub)], idx2d)

        def sbody(p, carry):
            j0 = p * 2
            cps = [
                pltpu.async_copy(
                    mout_hbm.at[pl.ds(base + (j0 + b) * SUB, SUB)],
                    mrow.at[b], sem)
                for b in range(2)
            ]
            for b in range(2):
                cps[b].wait()
                pltpu.sync_copy(mrow.at[b], shared.at[idx2d.at[j0 + b]],
                                add=True)
            return carry

        lax.fori_loop(0, nsub // 2, sbody, 0)
        plsc.subcore_barrier()

        pltpu.sync_copy(shared.at[pl.ds(sid * zrows, zrows)],
                        acc_out.at[cid, pl.ds(sid * zrows, zrows)])

    return sc_scatter_rows


def _make_sc_scatter_scalars(N_ACC, E_pad, EW):
    nvec = EW // 16

    @functools.partial(
        pl.kernel,
        out_type=[
            jax.ShapeDtypeStruct((NW * 4 * N_ACC,), F32),
        ],
        mesh=_sc_mesh(),
        scratch_types=[
            pltpu.VMEM((EW,), jnp.int32),
            pltpu.VMEM((EW,), F32),
            pltpu.VMEM((EW,), F32),
            pltpu.VMEM((EW,), F32),
            pltpu.VMEM((N_ACC,), F32),
            pltpu.VMEM((N_ACC,), F32),
            pltpu.VMEM((N_ACC,), F32),
            pltpu.VMEM((N_ACC,), F32),
        ],
        compiler_params=_SC_PARAMS,
    )
    def sc_scatter_scalars(dst_hbm, tx_hbm, ty_hbm, tz_hbm, sacc_out,
                           dstv, txv, tyv, tzv, ax, ay, az, ad):
        cid = lax.axis_index("c")
        sid = lax.axis_index("s")
        wid = sid * 2 + cid
        base = wid * EW

        z16 = jnp.zeros((16,), F32)

        def zbody(i, carry):
            off = i * 16
            ax[pl.ds(off, 16)] = z16
            ay[pl.ds(off, 16)] = z16
            az[pl.ds(off, 16)] = z16
            ad[pl.ds(off, 16)] = z16
            return carry

        lax.fori_loop(0, N_ACC // 16, zbody, 0)

        pltpu.sync_copy(dst_hbm.at[pl.ds(base, EW)], dstv)
        pltpu.sync_copy(tx_hbm.at[pl.ds(base, EW)], txv)
        pltpu.sync_copy(ty_hbm.at[pl.ds(base, EW)], tyv)
        pltpu.sync_copy(tz_hbm.at[pl.ds(base, EW)], tzv)

        one16 = jnp.ones((16,), F32)

        def vbody(k, carry):
            off = k * 16
            d16 = dstv[pl.ds(off, 16)]
            plsc.addupdate_scatter(ax, [d16], txv[pl.ds(off, 16)])
            plsc.addupdate_scatter(ay, [d16], tyv[pl.ds(off, 16)])
            plsc.addupdate_scatter(az, [d16], tzv[pl.ds(off, 16)])
            plsc.addupdate_scatter(ad, [d16], one16)
            return carry

        lax.fori_loop(0, nvec, vbody, 0)

        for c, buf in ((0, ax), (1, ay), (2, az), (3, ad)):
            pltpu.sync_copy(
                buf, sacc_out.at[pl.ds((wid * 4 + c) * N_ACC, N_ACC)])

    return sc_scatter_scalars


def _edge_block_kernel(hs, hd, dx, dy, dz,
                       w1ie, w1io, w1je, w1jo, w0, b1, w2, b2, wc1, bc1, wc2,
                       mout, tx, ty, tz):
    dxr = dx[0]
    dyr = dy[0]
    dzr = dz[0]
    l2 = jnp.sqrt(dxr * dxr + dyr * dyr + dzr * dzr + 1e-12)
    us = hs[...]
    ud = hd[...]
    hi16 = jnp.uint32(0xFFFF0000)
    BF = jnp.bfloat16
    hse = lax.bitcast_convert_type(us << 16, F32).astype(BF)
    hso = lax.bitcast_convert_type(us & hi16, F32).astype(BF)
    hde = lax.bitcast_convert_type(ud << 16, F32).astype(BF)
    hdo = lax.bitcast_convert_type(ud & hi16, F32).astype(BF)
    dot = functools.partial(jnp.dot, preferred_element_type=F32)
    pre1 = (dot(hse, w1ie[...].astype(BF))
            + dot(hso, w1io[...].astype(BF))
            + dot(hde, w1je[...].astype(BF))
            + dot(hdo, w1jo[...].astype(BF))
            + lax.dot_general(l2, w0[...], (((0,), (0,)), ((), ())))
            + b1[...])
    t1 = jnp.tanh(pre1).astype(BF)
    mij = dot(t1, w2[...].astype(BF)) + b2[...]
    a1 = jnp.tanh(dot(mij.astype(BF), wc1[...].astype(BF)) + bc1[...])
    att = jnp.tanh(lax.dot_general(
        wc2[...].astype(BF), a1.astype(BF), (((0,), (1,)), ((), ())),
        preferred_element_type=F32))
    mout[...] = mij
    tx[0] = dxr * att
    ty[0] = dyr * att
    tz[0] = dzr * att


def _node_block_kernel(hid, a0, a1, wh1a, wh1b, bh1, wh2, bh2, hout):
    mi = a0[...] + a1[...]
    pre = (jnp.dot(hid[...], wh1a[...]) + jnp.dot(mi, wh1b[...]) + bh1[...])
    hout[...] = hid[...] + jnp.dot(jnp.tanh(pre), wh2[...]) + bh2[...]


def _coords_kernel(sacc, ct, cout):
    s = jnp.sum(sacc[...], axis=0)
    deg = jnp.maximum(s[3:4, :], 1.0)
    cout[...] = ct[...] + s[0:3, :] / deg


def kernel(coords, hidden, edges, W_m1, b_m1, W_m2, b_m2,
           W_c1, b_c1, W_c2, W_h1, b_h1, W_h2, b_h2):
    N, HD = hidden.shape
    E = edges.shape[1]
    MD = W_m2.shape[0]

    quantum = NW * SUB * BE // math.gcd(NW * SUB, BE)
    E_pad = -(-E // quantum) * quantum
    EW = E_pad // NW
    N_ACC = -(-(N + 1) // 128) * 128

    src = edges[0]
    dst = edges[1]
    pad = E_pad - E
    src_p = jnp.concatenate([src, jnp.zeros((pad,), jnp.int32)])
    dst_p = jnp.concatenate([dst, jnp.full((pad,), N, jnp.int32)])
    dst2d = dst_p.reshape(E_pad // SUB, SUB)

    HP = HD // 2
    hu = lax.bitcast_convert_type(hidden.astype(jnp.bfloat16), jnp.uint16)
    hpk = (hu[:, 0::2].astype(jnp.uint32)
           | (hu[:, 1::2].astype(jnp.uint32) << 16))
    hid_pad = jnp.pad(hpk, ((0, N_ACC - N), (0, 0)))
    cpad = jnp.pad(coords, ((0, N_ACC - N), (0, 1)))
    cflat = cpad.reshape(N_ACC * 4)
    zeros2d = jnp.zeros((N_ACC, HD), F32)

    dx, dy, dz = _make_sc_coorddiff(N_ACC, E_pad, EW)(cflat, src_p, dst_p)
    EW0 = (E_pad // 16) * 3 // 4 // 1024 * 1024
    hs, hd = _make_sc_hgather(N_ACC, HP, E_pad, EW0)(hid_pad, src_p, dst_p)

    NB = E_pad // BE
    dx3 = dx.reshape(NB, 1, BE)
    dy3 = dy.reshape(NB, 1, BE)
    dz3 = dz.reshape(NB, 1, BE)

    w0 = W_m1[0:1, :]
    w1i = W_m1[1:1 + HD, :]
    w1j = W_m1[1 + HD:1 + 2 * HD, :]
    b1 = b_m1.reshape(1, MD)
    b2 = b_m2.reshape(1, MD)
    bc1 = b_c1.reshape(1, MD)

    full = lambda shape: pl.BlockSpec(shape, lambda i: (0,) * len(shape))
    eb = pl.BlockSpec((BE, HP), lambda i: (i, 0))
    sb = pl.BlockSpec((1, 1, BE), lambda i: (i, 0, 0))

    mout, tx3, ty3, tz3 = pl.pallas_call(
        _edge_block_kernel,
        grid=(NB,),
        in_specs=[eb, eb, sb, sb, sb,
                  full((HP, MD)), full((HP, MD)), full((HP, MD)),
                  full((HP, MD)), full((1, MD)),
                  full((1, MD)), full((MD, MD)), full((1, MD)),
                  full((MD, MD)), full((1, MD)), full((MD, 1))],
        out_specs=[pl.BlockSpec((BE, MD), lambda i: (i, 0)), sb, sb, sb],
        out_shape=[
            jax.ShapeDtypeStruct((E_pad, MD), F32),
            jax.ShapeDtypeStruct((NB, 1, BE), F32),
            jax.ShapeDtypeStruct((NB, 1, BE), F32),
            jax.ShapeDtypeStruct((NB, 1, BE), F32),
        ],
    )(hs, hd, dx3, dy3, dz3,
      w1i[0::2, :], w1i[1::2, :], w1j[0::2, :], w1j[1::2, :],
      w0, b1, W_m2, b2, W_c1, bc1, W_c2)

    tx = tx3.reshape(E_pad)
    ty = ty3.reshape(E_pad)
    tz = tz3.reshape(E_pad)

    (acc2,) = _make_sc_scatter_rows(N_ACC, MD, E_pad, EW)(
        mout, dst2d, zeros2d)
    (sacc,) = _make_sc_scatter_scalars(N_ACC, E_pad, EW)(
        dst_p, tx, ty, tz)

    a0 = acc2[0, :N]
    a1 = acc2[1, :N]
    wh1a = W_h1[:HD, :]
    wh1b = W_h1[HD:, :]
    bh1 = b_h1.reshape(1, MD)
    bh2 = b_h2.reshape(1, HD)

    BN = 2000
    nb = pl.BlockSpec((BN, HD), lambda i: (i, 0))
    hidden_out = pl.pallas_call(
        _node_block_kernel,
        grid=(N // BN,),
        in_specs=[nb, nb, nb, full((HD, MD)), full((MD, MD)),
                  full((1, MD)), full((MD, HD)), full((1, HD))],
        out_specs=nb,
        out_shape=jax.ShapeDtypeStruct((N, HD), F32),
    )(hidden, a0, a1, wh1a, wh1b, bh1, W_h2, bh2)

    sacc3 = sacc.reshape(NW, 4, N_ACC)
    ct = jnp.pad(coords, ((0, N_ACC - N), (0, 0))).T
    cout = pl.pallas_call(
        _coords_kernel,
        grid=(1,),
        in_specs=[pl.BlockSpec((NW, 4, N_ACC), lambda i: (0, 0, 0)),
                  pl.BlockSpec((3, N_ACC), lambda i: (0, 0))],
        out_specs=pl.BlockSpec((3, N_ACC), lambda i: (0, 0)),
        out_shape=jax.ShapeDtypeStruct((3, N_ACC), F32),
    )(sacc3, ct)
    coords_out = cout.T[:N]

    return (coords_out, hidden_out)
